# Initial kernel scaffold; baseline (speedup 1.0000x reference)
import jax
import jax.numpy as jnp
from jax.experimental import pallas as pl


def kernel(x, edge_index, edge_type, W1_rel, W1_root, b1, W2_rel, W2_root, b2, W3_rel, W3_root, b3, Wc1, bc1, Wc2, bc2, Wc3, bc3):
    raise NotImplementedError("write your pallas kernel here")


# speedup vs baseline: 4.6266x; 4.6266x over previous
import functools

import jax
import jax.numpy as jnp
from jax import lax
from jax.experimental import pallas as pl
from jax.experimental.pallas import tpu as pltpu
from jax.experimental.pallas import tpu_sc as plsc

N = 10000
E = 320000
IN_CH = 128
HID = 64
R = 3

f32 = jnp.float32
i32 = jnp.int32

NC = 2
NS = 16
NW = NC * NS
EPT = E // NW
CH = 80
NCHUNK = EPT // CH
ROWS_T = (R + 1) * N
ACC_PT = 600
ACC_XT = N - NS * ACC_PT
ZR = 200
CNT_PT = 1920
CNT_PAD = NS * CNT_PT

_mesh = plsc.VectorSubcoreMesh(core_axis_name="c", subcore_axis_name="s")
_sc_params = pltpu.CompilerParams(use_tc_tiling_on_sc=False)


def _count_body(et_hbm, dst_hbm, cntp, etb, dstb, sidxb, onesb, z1b, cnt_sh,
                sem):
    del sem
    c = lax.axis_index("c")
    s = lax.axis_index("s")
    wid = s * NC + c
    ebase = wid * EPT

    z16 = jnp.zeros((16,), f32)
    one16 = jnp.ones((16,), f32)
    for k in range(CH // 16):
        onesb[pl.ds(k * 16, 16)] = one16

    def zcntb(i, _):
        z1b[pl.ds(i * 16, 16)] = z16
        return 0

    lax.fori_loop(0, CNT_PT // 16, zcntb, 0)
    pltpu.sync_copy(z1b, cnt_sh.at[pl.ds(s * CNT_PT, CNT_PT)])
    plsc.subcore_barrier()

    def chunk(ci, _):
        off = ebase + ci * CH
        pltpu.sync_copy(et_hbm.at[pl.ds(off, CH)], etb)
        pltpu.sync_copy(dst_hbm.at[pl.ds(off, CH)], dstb)
        for k in range(CH // 16):
            sl = pl.ds(k * 16, 16)
            sidxb[sl] = dstb[sl] * R + etb[sl]
        pltpu.sync_copy(onesb, cnt_sh.at[sidxb], add=True)
        return 0

    lax.fori_loop(0, NCHUNK, chunk, 0)
    plsc.subcore_barrier()
    pltpu.sync_copy(cnt_sh.at[pl.ds(s * CNT_PT, CNT_PT)],
                    cntp.at[c, pl.ds(s * CNT_PT, CNT_PT)])


_count_call = pl.kernel(
    _count_body,
    out_type=jax.ShapeDtypeStruct((NC, CNT_PAD), f32),
    mesh=_mesh,
    compiler_params=_sc_params,
    scratch_types=[
        pltpu.VMEM((CH,), i32),
        pltpu.VMEM((CH,), i32),
        pltpu.VMEM((CH,), i32),
        pltpu.VMEM((CH,), f32),
        pltpu.VMEM((CNT_PT,), f32),
        pltpu.VMEM_SHARED((CNT_PAD,), f32),
        pltpu.SemaphoreType.DMA,
    ],
)


def _zero_acc(s, zb, acc_sh):
    def zacc(j, _):
        pltpu.sync_copy(zb, acc_sh.at[pl.ds(s * ACC_PT + j * ZR, ZR)])
        return 0

    lax.fori_loop(0, ACC_PT // ZR, zacc, 0)

    @pl.when(s == 0)
    def _():
        def zext(j, _):
            pltpu.sync_copy(zb, acc_sh.at[pl.ds(NS * ACC_PT + j * ZR, ZR)])
            return 0

        lax.fori_loop(0, ACC_XT // ZR, zext, 0)


def _wb_acc(c, s, acc_sh, accp):
    def wb(j, _):
        r0 = s * ACC_PT + j * ZR
        pltpu.sync_copy(acc_sh.at[pl.ds(r0, ZR)], accp.at[c, pl.ds(r0, ZR)])
        return 0

    lax.fori_loop(0, ACC_PT // ZR, wb, 0)

    @pl.when(s == 0)
    def _():
        def wbe(j, _):
            r0 = NS * ACC_PT + j * ZR
            pltpu.sync_copy(acc_sh.at[pl.ds(r0, ZR)],
                            accp.at[c, pl.ds(r0, ZR)])
            return 0

        lax.fori_loop(0, ACC_XT // ZR, wbe, 0)


def _scatter_body(first, *refs):
    if first:
        (t_hbm, et_hbm, src_hbm, dst_hbm, winv_hbm, accp, wout,
         etb, srcb, dstb, gidxb, sidxb, wb_, msgb, zb, acc_sh, sem) = refs
    else:
        (t_hbm, et_hbm, src_hbm, dst_hbm, w_hbm, accp,
         etb, srcb, dstb, gidxb, sidxb, wb_, msgb, zb, acc_sh, sem) = refs
    c = lax.axis_index("c")
    s = lax.axis_index("s")
    wid = s * NC + c
    ebase = wid * EPT

    z16 = jnp.zeros((16,), f32)

    def zrow(i, _):
        for j in range(HID // 16):
            zb[i, pl.ds(j * 16, 16)] = z16
        return 0

    lax.fori_loop(0, ZR, zrow, 0)
    _zero_acc(s, zb, acc_sh)
    plsc.subcore_barrier()

    def chunk(ci, _):
        off = ebase + ci * CH
        pltpu.sync_copy(et_hbm.at[pl.ds(off, CH)], etb)
        pltpu.sync_copy(src_hbm.at[pl.ds(off, CH)], srcb)
        pltpu.sync_copy(dst_hbm.at[pl.ds(off, CH)], dstb.at[0])
        for k in range(CH // 16):
            sl = pl.ds(k * 16, 16)
            e16 = etb[sl]
            gidxb[sl] = srcb[sl] * (R + 1) + e16
            if first:
                sidxb[sl] = dstb[0, sl] * R + e16
        if first:
            pltpu.async_copy(winv_hbm.at[sidxb], wb_, sem).wait()
            pltpu.sync_copy(wb_, wout.at[pl.ds(off, CH)])
        else:
            pltpu.sync_copy(w_hbm.at[pl.ds(off, CH)], wb_)
        pltpu.async_copy(t_hbm.at[gidxb], msgb, sem).wait()
        for g in range(CH // 16):
            w16 = wb_[pl.ds(g * 16, 16)]
            for m in range(16):
                k = g * 16 + m
                wk = w16[m]
                for j in range(HID // 16):
                    sl = pl.ds(j * 16, 16)
                    msgb[k, sl] = msgb[k, sl] * wk
        pltpu.sync_copy(msgb, acc_sh.at[dstb.at[0]], add=True)
        return 0

    lax.fori_loop(0, NCHUNK, chunk, 0)
    plsc.subcore_barrier()
    _wb_acc(c, s, acc_sh, accp)


def _scatter_scratch():
    return [
        pltpu.VMEM((CH,), i32),
        pltpu.VMEM((CH,), i32),
        pltpu.VMEM((1, CH), i32),
        pltpu.VMEM((CH,), i32),
        pltpu.VMEM((CH,), i32),
        pltpu.VMEM((CH,), f32),
        pltpu.VMEM((CH, HID), f32),
        pltpu.VMEM((ZR, HID), f32),
        pltpu.VMEM_SHARED((N, HID), f32),
        pltpu.SemaphoreType.DMA,
    ]


_scatter_first = pl.kernel(
    functools.partial(_scatter_body, True),
    out_type=[jax.ShapeDtypeStruct((NC, N, HID), f32),
              jax.ShapeDtypeStruct((E,), f32)],
    mesh=_mesh,
    compiler_params=_sc_params,
    scratch_types=_scatter_scratch(),
)

_scatter_rest = pl.kernel(
    functools.partial(_scatter_body, False),
    out_type=jax.ShapeDtypeStruct((NC, N, HID), f32),
    mesh=_mesh,
    compiler_params=_sc_params,
    scratch_types=_scatter_scratch(),
)


def _edge_gather_body(a_hbm, b_hbm, src_hbm, dst_hbm, ga_hbm, gb_hbm,
                      srcb, dstb, gab, gbb, sema, semb):
    c = lax.axis_index("c")
    s = lax.axis_index("s")
    wid = s * NC + c
    ebase = wid * EPT

    def chunk(ci, _):
        off = ebase + ci * CH
        pltpu.sync_copy(src_hbm.at[pl.ds(off, CH)], srcb)
        pltpu.sync_copy(dst_hbm.at[pl.ds(off, CH)], dstb)
        cpa = pltpu.async_copy(a_hbm.at[srcb], gab, sema)
        cpb = pltpu.async_copy(b_hbm.at[dstb], gbb, semb)
        cpa.wait()
        cpb.wait()
        pltpu.sync_copy(gab, ga_hbm.at[pl.ds(off, CH)])
        pltpu.sync_copy(gbb, gb_hbm.at[pl.ds(off, CH)])
        return 0

    lax.fori_loop(0, NCHUNK, chunk, 0)


_edge_gather = pl.kernel(
    _edge_gather_body,
    out_type=[jax.ShapeDtypeStruct((E, HID), f32),
              jax.ShapeDtypeStruct((E, HID), f32)],
    mesh=_mesh,
    compiler_params=_sc_params,
    scratch_types=[
        pltpu.VMEM((CH,), i32),
        pltpu.VMEM((CH,), i32),
        pltpu.VMEM((CH, HID), f32),
        pltpu.VMEM((CH, HID), f32),
        pltpu.SemaphoreType.DMA,
        pltpu.SemaphoreType.DMA,
    ],
)


def _inv_body(c_ref, o_ref):
    cc = c_ref[...]
    o_ref[...] = 1.0 / jnp.maximum(cc[0] + cc[1], 1.0)


def _inv_call(cntp):
    cc = cntp.reshape(NC, CNT_PAD // 128, 128)
    return pl.pallas_call(
        _inv_body,
        out_shape=jax.ShapeDtypeStruct((CNT_PAD // 128, 128), f32),
    )(cc).reshape(CNT_PAD)


def _mm_body(x_ref, w_ref, b_ref, o_ref):
    o_ref[...] = (
        jnp.dot(x_ref[...], w_ref[...], preferred_element_type=f32)
        + b_ref[...]
    )


def _mm(xx, w, b, bn):
    m, k = xx.shape
    return pl.pallas_call(
        _mm_body,
        grid=(m // bn,),
        in_specs=[
            pl.BlockSpec((bn, k), lambda i: (i, 0)),
            pl.BlockSpec(w.shape, lambda i: (0, 0)),
            pl.BlockSpec(b.shape, lambda i: (0, 0)),
        ],
        out_specs=pl.BlockSpec((bn, w.shape[1]), lambda i: (i, 0)),
        out_shape=jax.ShapeDtypeStruct((m, w.shape[1]), f32),
    )(xx, w, b)


def _agg(t_ref, p0_ref, p1_ref):
    root = t_ref[...][:, R * HID:]
    return root + p0_ref[...] + p1_ref[...]


def _combine_mm_body(t_ref, p0_ref, p1_ref, w_ref, b_ref, xn_ref, tn_ref):
    x = jnp.maximum(_agg(t_ref, p0_ref, p1_ref), 0.0)
    xn_ref[...] = x
    tn_ref[...] = (
        jnp.dot(x, w_ref[...], preferred_element_type=f32) + b_ref[...]
    )


def _combine_mm(t, p0, p1, w, b, bn):
    return pl.pallas_call(
        _combine_mm_body,
        grid=(N // bn,),
        in_specs=[
            pl.BlockSpec((bn, (R + 1) * HID), lambda i: (i, 0)),
            pl.BlockSpec((bn, HID), lambda i: (i, 0)),
            pl.BlockSpec((bn, HID), lambda i: (i, 0)),
            pl.BlockSpec(w.shape, lambda i: (0, 0)),
            pl.BlockSpec(b.shape, lambda i: (0, 0)),
        ],
        out_specs=[
            pl.BlockSpec((bn, HID), lambda i: (i, 0)),
            pl.BlockSpec((bn, w.shape[1]), lambda i: (i, 0)),
        ],
        out_shape=[
            jax.ShapeDtypeStruct((N, HID), f32),
            jax.ShapeDtypeStruct((N, w.shape[1]), f32),
        ],
    )(t, p0, p1, w, b)


def _combine3_body(t_ref, p0_ref, p1_ref, x1_ref, w_ref, b_ref,
                   a_ref, bo_ref):
    x3 = _agg(t_ref, p0_ref, p1_ref) + x1_ref[...]
    ab = jnp.dot(x3, w_ref[...], preferred_element_type=f32) + b_ref[...]
    a_ref[...] = ab[:, :HID]
    bo_ref[...] = ab[:, HID:]


def _combine3(t, p0, p1, x1, w, b, bn):
    return pl.pallas_call(
        _combine3_body,
        grid=(N // bn,),
        in_specs=[
            pl.BlockSpec((bn, (R + 1) * HID), lambda i: (i, 0)),
            pl.BlockSpec((bn, HID), lambda i: (i, 0)),
            pl.BlockSpec((bn, HID), lambda i: (i, 0)),
            pl.BlockSpec((bn, HID), lambda i: (i, 0)),
            pl.BlockSpec(w.shape, lambda i: (0, 0)),
            pl.BlockSpec(b.shape, lambda i: (0, 0)),
        ],
        out_specs=[
            pl.BlockSpec((bn, HID), lambda i: (i, 0)),
            pl.BlockSpec((bn, HID), lambda i: (i, 0)),
        ],
        out_shape=[
            jax.ShapeDtypeStruct((N, HID), f32),
            jax.ShapeDtypeStruct((N, HID), f32),
        ],
    )(t, p0, p1, x1, w, b)


def _edge_mlp_body(ga_ref, gb_ref, w2_ref, b2_ref, w3_ref, b3_ref, o_ref):
    h = jnp.maximum(ga_ref[...] + gb_ref[...], 0.0)
    h2 = jnp.maximum(
        jnp.dot(h, w2_ref[...], preferred_element_type=f32) + b2_ref[...], 0.0
    )
    o_ref[...] = (
        jnp.dot(h2, w3_ref[...], preferred_element_type=f32) + b3_ref[...]
    )


def _edge_mlp(ga, gb, w2, b2, w3, b3, be):
    return pl.pallas_call(
        _edge_mlp_body,
        grid=(E // be,),
        in_specs=[
            pl.BlockSpec((be, HID), lambda i: (i, 0)),
            pl.BlockSpec((be, HID), lambda i: (i, 0)),
            pl.BlockSpec(w2.shape, lambda i: (0, 0)),
            pl.BlockSpec(b2.shape, lambda i: (0, 0)),
            pl.BlockSpec(w3.shape, lambda i: (0, 0)),
            pl.BlockSpec(b3.shape, lambda i: (0, 0)),
        ],
        out_specs=pl.BlockSpec((be, 3), lambda i: (i, 0)),
        out_shape=jax.ShapeDtypeStruct((E, 3), f32),
    )(ga, gb, w2, b2, w3, b3)


def kernel(x, edge_index, edge_type, W1_rel, W1_root, b1, W2_rel, W2_root,
           b2, W3_rel, W3_root, b3, Wc1, bc1, Wc2, bc2, Wc3, bc3):
    src = edge_index[0]
    dst = edge_index[1]

    def wcat(w_rel, w_root, b):
        w = jnp.concatenate([w_rel[0], w_rel[1], w_rel[2], w_root], axis=1)
        bias = jnp.concatenate([jnp.zeros((R * HID,), f32), b])[None]
        return w, bias

    w1, bias1 = wcat(W1_rel, W1_root, b1)
    w2, bias2 = wcat(W2_rel, W2_root, b2)
    w3, bias3 = wcat(W3_rel, W3_root, b3)
    wab = jnp.concatenate([Wc1[:HID], Wc1[HID:]], axis=1)
    bab = jnp.concatenate([bc1, jnp.zeros((HID,), f32)])[None]

    bn = 1000
    cntp = _count_call(edge_type, dst)
    winv = _inv_call(cntp)

    t1 = _mm(x, w1, bias1, bn)
    p1, wedge = _scatter_first(t1.reshape(ROWS_T, HID), edge_type, src, dst,
                               winv)
    x1, t2 = _combine_mm(t1, p1[0], p1[1], w2, bias2, bn)
    p2 = _scatter_rest(t2.reshape(ROWS_T, HID), edge_type, src, dst, wedge)
    _, t3 = _combine_mm(t2, p2[0], p2[1], w3, bias3, bn)
    p3 = _scatter_rest(t3.reshape(ROWS_T, HID), edge_type, src, dst, wedge)
    a, bnode = _combine3(t3, p3[0], p3[1], x1, wab, bab, bn)

    ga, gb = _edge_gather(a, bnode, src, dst)
    return _edge_mlp(ga, gb, Wc2, bc2[None], Wc3, bc3[None], 2000)


# speedup vs baseline: 9.0394x; 1.9538x over previous
import jax
import jax.numpy as jnp
from jax import lax
from jax.experimental import pallas as pl
from jax.experimental.pallas import tpu as pltpu
from jax.experimental.pallas import tpu_sc as plsc

N = 10000
E = 320000
IN_CH = 128
HID = 64
R = 3

f32 = jnp.float32
i32 = jnp.int32

NC = 2
NS = 16
NW = NC * NS
EPT = E // NW
CH = 80
NCHUNK = EPT // CH
EPS = E // NS
NCHUNK_C = EPS // CH
ROWS_T = (R + 1) * N
ACC_PT = 600
ACC_XT = N - NS * ACC_PT
ZR = 200
CNT_PT = 1920
CNT_PAD = NS * CNT_PT

_mesh = plsc.VectorSubcoreMesh(core_axis_name="c", subcore_axis_name="s")
_sc_params = pltpu.CompilerParams(use_tc_tiling_on_sc=False)


def _prep_body(et_hbm, dst_hbm, w_hbm,
               eb, db, sidxb, onesb, cbuf, wbuf, cnt_sh, lsem, ssem, osem):
    s = lax.axis_index("s")
    c = lax.axis_index("c")
    wid = s * NC + c

    z16 = jnp.zeros((16,), f32)
    one16 = jnp.ones((16,), f32)
    for k in range(CH // 16):
        onesb[pl.ds(k * 16, 16)] = one16

    def zcb(i, _):
        cbuf[pl.ds(i * 16, 16)] = z16
        return 0

    lax.fori_loop(0, CNT_PT // 16, zcb, 0)
    pltpu.sync_copy(cbuf, cnt_sh.at[pl.ds(s * CNT_PT, CNT_PT)])
    plsc.subcore_barrier()

    base1 = s * EPS

    def issue_loads1(ci, b):
        off = base1 + ci * CH
        pltpu.async_copy(et_hbm.at[pl.ds(off, CH)], eb.at[b], lsem)
        pltpu.async_copy(dst_hbm.at[pl.ds(off, CH)], db.at[b], lsem)

    def wait_loads():
        pltpu.make_async_copy(et_hbm.at[pl.ds(0, CH)], eb.at[0], lsem).wait()
        pltpu.make_async_copy(dst_hbm.at[pl.ds(0, CH)], db.at[0], lsem).wait()

    def wait_scat():
        pltpu.make_async_copy(
            onesb, cnt_sh.at[pl.ds(0, CH)], ssem).wait()

    issue_loads1(0, 0)

    def body1(i, _):
        bi = i % 2
        bn = 1 - bi
        wait_loads()

        @pl.when(i >= 2)
        def _():
            wait_scat()

        for k in range(CH // 16):
            sl = pl.ds(k * 16, 16)
            sidxb[bi, sl] = db[bi, sl] * R + eb[bi, sl]
        pltpu.async_copy(onesb, cnt_sh.at[sidxb.at[bi]], ssem, add=True)
        nx = jnp.minimum(i + 1, NCHUNK_C - 1)
        issue_loads1(nx, bn)
        return 0

    lax.fori_loop(0, NCHUNK_C, body1, 0)
    wait_scat()
    wait_scat()
    wait_loads()
    plsc.subcore_barrier()

    pltpu.sync_copy(cnt_sh.at[pl.ds(s * CNT_PT, CNT_PT)], cbuf)

    def invb(i, _):
        sl = pl.ds(i * 16, 16)
        cbuf[sl] = 1.0 / jnp.maximum(cbuf[sl], 1.0)
        return 0

    lax.fori_loop(0, CNT_PT // 16, invb, 0)
    pltpu.sync_copy(cbuf, cnt_sh.at[pl.ds(s * CNT_PT, CNT_PT)])
    plsc.subcore_barrier()

    base3 = wid * EPT

    def issue_loads3(ci, b):
        off = base3 + ci * CH
        pltpu.async_copy(et_hbm.at[pl.ds(off, CH)], eb.at[b], lsem)
        pltpu.async_copy(dst_hbm.at[pl.ds(off, CH)], db.at[b], lsem)

    issue_loads3(0, 0)

    def body3(i, _):
        bi = i % 2
        bn = 1 - bi
        wait_loads()
        for k in range(CH // 16):
            sl = pl.ds(k * 16, 16)
            sidxb[bi, sl] = db[bi, sl] * R + eb[bi, sl]

        @pl.when(i >= 2)
        def _():
            pltpu.make_async_copy(
                wbuf.at[0], w_hbm.at[pl.ds(0, CH)], osem).wait()

        pltpu.sync_copy(cnt_sh.at[sidxb.at[bi]], wbuf.at[bi])
        off = base3 + i * CH
        pltpu.async_copy(wbuf.at[bi], w_hbm.at[pl.ds(off, CH)], osem)
        nx = jnp.minimum(i + 1, NCHUNK - 1)
        issue_loads3(nx, bn)
        return 0

    lax.fori_loop(0, NCHUNK, body3, 0)
    for _ in range(2):
        pltpu.make_async_copy(
            wbuf.at[0], w_hbm.at[pl.ds(0, CH)], osem).wait()
    wait_loads()


_prep_call = pl.kernel(
    _prep_body,
    out_type=jax.ShapeDtypeStruct((E,), f32),
    mesh=_mesh,
    compiler_params=_sc_params,
    scratch_types=[
        pltpu.VMEM((2, CH), i32),
        pltpu.VMEM((2, CH), i32),
        pltpu.VMEM((2, CH), i32),
        pltpu.VMEM((CH,), f32),
        pltpu.VMEM((CNT_PT,), f32),
        pltpu.VMEM((2, CH), f32),
        pltpu.VMEM_SHARED((CNT_PAD,), f32),
        pltpu.SemaphoreType.DMA,
        pltpu.SemaphoreType.DMA,
        pltpu.SemaphoreType.DMA,
    ],
)


def _zero_acc(s, zb, acc_sh):
    def zacc(j, _):
        pltpu.sync_copy(zb, acc_sh.at[pl.ds(s * ACC_PT + j * ZR, ZR)])
        return 0

    lax.fori_loop(0, ACC_PT // ZR, zacc, 0)

    @pl.when(s == 0)
    def _():
        def zext(j, _):
            pltpu.sync_copy(zb, acc_sh.at[pl.ds(NS * ACC_PT + j * ZR, ZR)])
            return 0

        lax.fori_loop(0, ACC_XT // ZR, zext, 0)


def _wb_acc(c, s, acc_sh, accp):
    def wb(j, _):
        r0 = s * ACC_PT + j * ZR
        pltpu.sync_copy(acc_sh.at[pl.ds(r0, ZR)], accp.at[c, pl.ds(r0, ZR)])
        return 0

    lax.fori_loop(0, ACC_PT // ZR, wb, 0)

    @pl.when(s == 0)
    def _():
        def wbe(j, _):
            r0 = NS * ACC_PT + j * ZR
            pltpu.sync_copy(acc_sh.at[pl.ds(r0, ZR)],
                            accp.at[c, pl.ds(r0, ZR)])
            return 0

        lax.fori_loop(0, ACC_XT // ZR, wbe, 0)


def _scatter_body(t_hbm, et_hbm, src_hbm, dst_hbm, w_hbm, accp,
                  eb, sb, db, wbuf, gidxb, msgb, zb, acc_sh,
                  lsem, gsem, ssem):
    c = lax.axis_index("c")
    s = lax.axis_index("s")
    wid = s * NC + c
    ebase = wid * EPT

    z16 = jnp.zeros((16,), f32)

    def zrow(i, _):
        for j in range(HID // 16):
            zb[i, pl.ds(j * 16, 16)] = z16
        return 0

    lax.fori_loop(0, ZR, zrow, 0)
    _zero_acc(s, zb, acc_sh)
    plsc.subcore_barrier()

    def issue_loads(ci, b):
        off = ebase + ci * CH
        pltpu.async_copy(et_hbm.at[pl.ds(off, CH)], eb.at[b], lsem)
        pltpu.async_copy(src_hbm.at[pl.ds(off, CH)], sb.at[b], lsem)
        pltpu.async_copy(dst_hbm.at[pl.ds(off, CH)], db.at[b], lsem)
        pltpu.async_copy(w_hbm.at[pl.ds(off, CH)], wbuf.at[b], lsem)

    def wait_loads():
        pltpu.make_async_copy(et_hbm.at[pl.ds(0, CH)], eb.at[0], lsem).wait()
        pltpu.make_async_copy(src_hbm.at[pl.ds(0, CH)], sb.at[0], lsem).wait()
        pltpu.make_async_copy(dst_hbm.at[pl.ds(0, CH)], db.at[0], lsem).wait()
        pltpu.make_async_copy(
            w_hbm.at[pl.ds(0, CH)], wbuf.at[0], lsem).wait()

    def wait_scat():
        pltpu.make_async_copy(
            msgb.at[0], acc_sh.at[pl.ds(0, CH)], ssem).wait()

    def scale_and_scatter(bm, b3):
        for g in range(CH // 16):
            w16 = wbuf[b3, pl.ds(g * 16, 16)]
            for m in range(16):
                k = g * 16 + m
                wk = w16[m]
                for j in range(HID // 16):
                    sl = pl.ds(j * 16, 16)
                    msgb[bm, k, sl] = msgb[bm, k, sl] * wk
        pltpu.async_copy(msgb.at[bm], acc_sh.at[db.at[b3]], ssem, add=True)

    issue_loads(0, 0)

    def body(i, _):
        bi = i % 2
        bn = 1 - bi
        b3 = lax.rem(i, 3)
        p3 = lax.rem(i + 2, 3)
        wait_loads()
        for k in range(CH // 16):
            sl = pl.ds(k * 16, 16)
            gidxb[bi, sl] = sb[b3, sl] * (R + 1) + eb[b3, sl]

        @pl.when(i >= 2)
        def _():
            wait_scat()

        pltpu.async_copy(t_hbm.at[gidxb.at[bi]], msgb.at[bi], gsem)
        nx = jnp.minimum(i + 1, NCHUNK - 1)
        issue_loads(nx, lax.rem(i + 1, 3))

        @pl.when(i >= 1)
        def _():
            pltpu.make_async_copy(
                t_hbm.at[gidxb.at[bn]], msgb.at[bn], gsem).wait()
            scale_and_scatter(bn, p3)

        return 0

    lax.fori_loop(0, NCHUNK, body, 0)
    pltpu.make_async_copy(
        t_hbm.at[gidxb.at[(NCHUNK - 1) % 2]],
        msgb.at[(NCHUNK - 1) % 2], gsem).wait()
    scale_and_scatter((NCHUNK - 1) % 2, (NCHUNK - 1) % 3)
    wait_scat()
    wait_scat()
    wait_loads()
    plsc.subcore_barrier()
    _wb_acc(c, s, acc_sh, accp)


_scatter_call = pl.kernel(
    _scatter_body,
    out_type=jax.ShapeDtypeStruct((NC, N, HID), f32),
    mesh=_mesh,
    compiler_params=_sc_params,
    scratch_types=[
        pltpu.VMEM((3, CH), i32),
        pltpu.VMEM((3, CH), i32),
        pltpu.VMEM((3, CH), i32),
        pltpu.VMEM((3, CH), f32),
        pltpu.VMEM((2, CH), i32),
        pltpu.VMEM((2, CH, HID), f32),
        pltpu.VMEM((ZR, HID), f32),
        pltpu.VMEM_SHARED((N, HID), f32),
        pltpu.SemaphoreType.DMA,
        pltpu.SemaphoreType.DMA,
        pltpu.SemaphoreType.DMA,
    ],
)


def _edge_body(a_hbm, b_hbm, src_hbm, dst_hbm, h_hbm,
               sb, db, ga, gb, lsem, gsem, osem):
    c = lax.axis_index("c")
    s = lax.axis_index("s")
    wid = s * NC + c
    ebase = wid * EPT

    def issue_loads(ci, b):
        off = ebase + ci * CH
        pltpu.async_copy(src_hbm.at[pl.ds(off, CH)], sb.at[b], lsem)
        pltpu.async_copy(dst_hbm.at[pl.ds(off, CH)], db.at[b], lsem)

    def wait_loads():
        pltpu.make_async_copy(src_hbm.at[pl.ds(0, CH)], sb.at[0], lsem).wait()
        pltpu.make_async_copy(dst_hbm.at[pl.ds(0, CH)], db.at[0], lsem).wait()

    def wait_gath():
        pltpu.make_async_copy(a_hbm.at[sb.at[0]], ga.at[0], gsem).wait()
        pltpu.make_async_copy(b_hbm.at[db.at[0]], gb.at[0], gsem).wait()

    def wait_out():
        pltpu.make_async_copy(
            ga.at[0], h_hbm.at[pl.ds(0, CH)], osem).wait()

    def combine_and_write(i, b):
        for k in range(CH):
            for j in range(HID // 16):
                sl = pl.ds(j * 16, 16)
                ga[b, k, sl] = jnp.maximum(ga[b, k, sl] + gb[b, k, sl], 0.0)
        off = ebase + i * CH
        pltpu.async_copy(ga.at[b], h_hbm.at[pl.ds(off, CH)], osem)

    issue_loads(0, 0)

    def body(i, _):
        bi = i % 2
        bn = 1 - bi
        wait_loads()

        @pl.when(i >= 2)
        def _():
            wait_out()

        pltpu.async_copy(a_hbm.at[sb.at[bi]], ga.at[bi], gsem)
        pltpu.async_copy(b_hbm.at[db.at[bi]], gb.at[bi], gsem)

        @pl.when(i >= 1)
        def _():
            wait_gath()

        nx = jnp.minimum(i + 1, NCHUNK - 1)
        issue_loads(nx, bn)

        @pl.when(i >= 1)
        def _():
            combine_and_write(i - 1, bn)

        return 0

    lax.fori_loop(0, NCHUNK, body, 0)
    blast = (NCHUNK - 1) % 2
    wait_gath()
    combine_and_write(NCHUNK - 1, blast)
    wait_out()
    wait_out()
    wait_loads()


_edge_call = pl.kernel(
    _edge_body,
    out_type=jax.ShapeDtypeStruct((E, HID), f32),
    mesh=_mesh,
    compiler_params=_sc_params,
    scratch_types=[
        pltpu.VMEM((2, CH), i32),
        pltpu.VMEM((2, CH), i32),
        pltpu.VMEM((2, CH, HID), f32),
        pltpu.VMEM((2, CH, HID), f32),
        pltpu.SemaphoreType.DMA,
        pltpu.SemaphoreType.DMA,
        pltpu.SemaphoreType.DMA,
    ],
)


def _mm_body(x_ref, w_ref, b_ref, o_ref):
    o_ref[...] = (
        jnp.dot(x_ref[...], w_ref[...], preferred_element_type=f32)
        + b_ref[...]
    )


def _mm(xx, w, b, bn):
    m, k = xx.shape
    return pl.pallas_call(
        _mm_body,
        grid=(m // bn,),
        in_specs=[
            pl.BlockSpec((bn, k), lambda i: (i, 0)),
            pl.BlockSpec(w.shape, lambda i: (0, 0)),
            pl.BlockSpec(b.shape, lambda i: (0, 0)),
        ],
        out_specs=pl.BlockSpec((bn, w.shape[1]), lambda i: (i, 0)),
        out_shape=jax.ShapeDtypeStruct((m, w.shape[1]), f32),
    )(xx, w, b)


def _agg(t_ref, p0_ref, p1_ref):
    root = t_ref[...][:, R * HID:]
    return root + p0_ref[...] + p1_ref[...]


def _combine_mm_body(t_ref, p0_ref, p1_ref, w_ref, b_ref, xn_ref, tn_ref):
    x = jnp.maximum(_agg(t_ref, p0_ref, p1_ref), 0.0)
    xn_ref[...] = x
    tn_ref[...] = (
        jnp.dot(x, w_ref[...], preferred_element_type=f32) + b_ref[...]
    )


def _combine_mm(t, p0, p1, w, b, bn):
    return pl.pallas_call(
        _combine_mm_body,
        grid=(N // bn,),
        in_specs=[
            pl.BlockSpec((bn, (R + 1) * HID), lambda i: (i, 0)),
            pl.BlockSpec((bn, HID), lambda i: (i, 0)),
            pl.BlockSpec((bn, HID), lambda i: (i, 0)),
            pl.BlockSpec(w.shape, lambda i: (0, 0)),
            pl.BlockSpec(b.shape, lambda i: (0, 0)),
        ],
        out_specs=[
            pl.BlockSpec((bn, HID), lambda i: (i, 0)),
            pl.BlockSpec((bn, w.shape[1]), lambda i: (i, 0)),
        ],
        out_shape=[
            jax.ShapeDtypeStruct((N, HID), f32),
            jax.ShapeDtypeStruct((N, w.shape[1]), f32),
        ],
    )(t, p0, p1, w, b)


def _combine3_body(t_ref, p0_ref, p1_ref, x1_ref, w_ref, b_ref,
                   a_ref, bo_ref):
    x3 = _agg(t_ref, p0_ref, p1_ref) + x1_ref[...]
    ab = jnp.dot(x3, w_ref[...], preferred_element_type=f32) + b_ref[...]
    a_ref[...] = ab[:, :HID]
    bo_ref[...] = ab[:, HID:]


def _combine3(t, p0, p1, x1, w, b, bn):
    return pl.pallas_call(
        _combine3_body,
        grid=(N // bn,),
        in_specs=[
            pl.BlockSpec((bn, (R + 1) * HID), lambda i: (i, 0)),
            pl.BlockSpec((bn, HID), lambda i: (i, 0)),
            pl.BlockSpec((bn, HID), lambda i: (i, 0)),
            pl.BlockSpec((bn, HID), lambda i: (i, 0)),
            pl.BlockSpec(w.shape, lambda i: (0, 0)),
            pl.BlockSpec(b.shape, lambda i: (0, 0)),
        ],
        out_specs=[
            pl.BlockSpec((bn, HID), lambda i: (i, 0)),
            pl.BlockSpec((bn, HID), lambda i: (i, 0)),
        ],
        out_shape=[
            jax.ShapeDtypeStruct((N, HID), f32),
            jax.ShapeDtypeStruct((N, HID), f32),
        ],
    )(t, p0, p1, x1, w, b)


def _edge_mlp_body(h_ref, w2_ref, b2_ref, w3_ref, b3_ref, o_ref):
    h2 = jnp.maximum(
        jnp.dot(h_ref[...], w2_ref[...], preferred_element_type=f32)
        + b2_ref[...], 0.0
    )
    o_ref[...] = (
        jnp.dot(h2, w3_ref[...], preferred_element_type=f32) + b3_ref[...]
    )


def _edge_mlp(h, w2, b2, w3, b3, be):
    return pl.pallas_call(
        _edge_mlp_body,
        grid=(E // be,),
        in_specs=[
            pl.BlockSpec((be, HID), lambda i: (i, 0)),
            pl.BlockSpec(w2.shape, lambda i: (0, 0)),
            pl.BlockSpec(b2.shape, lambda i: (0, 0)),
            pl.BlockSpec(w3.shape, lambda i: (0, 0)),
            pl.BlockSpec(b3.shape, lambda i: (0, 0)),
        ],
        out_specs=pl.BlockSpec((be, 3), lambda i: (i, 0)),
        out_shape=jax.ShapeDtypeStruct((E, 3), f32),
    )(h, w2, b2, w3, b3)


def kernel(x, edge_index, edge_type, W1_rel, W1_root, b1, W2_rel, W2_root,
           b2, W3_rel, W3_root, b3, Wc1, bc1, Wc2, bc2, Wc3, bc3):
    src = edge_index[0]
    dst = edge_index[1]

    def wcat(w_rel, w_root, b):
        w = jnp.concatenate([w_rel[0], w_rel[1], w_rel[2], w_root], axis=1)
        bias = jnp.concatenate([jnp.zeros((R * HID,), f32), b])[None]
        return w, bias

    w1, bias1 = wcat(W1_rel, W1_root, b1)
    w2, bias2 = wcat(W2_rel, W2_root, b2)
    w3, bias3 = wcat(W3_rel, W3_root, b3)
    wab = jnp.concatenate([Wc1[:HID], Wc1[HID:]], axis=1)
    bab = jnp.concatenate([bc1, jnp.zeros((HID,), f32)])[None]

    bn = 1000
    wedge = _prep_call(edge_type, dst)

    t1 = _mm(x, w1, bias1, bn)
    p1 = _scatter_call(t1.reshape(ROWS_T, HID), edge_type, src, dst, wedge)
    x1, t2 = _combine_mm(t1, p1[0], p1[1], w2, bias2, bn)
    p2 = _scatter_call(t2.reshape(ROWS_T, HID), edge_type, src, dst, wedge)
    _, t3 = _combine_mm(t2, p2[0], p2[1], w3, bias3, bn)
    p3 = _scatter_call(t3.reshape(ROWS_T, HID), edge_type, src, dst, wedge)
    a, bnode = _combine3(t3, p3[0], p3[1], x1, wab, bab, bn)

    h = _edge_call(a, bnode, src, dst)
    return _edge_mlp(h, Wc2, bc2[None], Wc3, bc3[None], 2000)


# speedup vs baseline: 10.1244x; 1.1200x over previous
import jax
import jax.numpy as jnp
from jax import lax
from jax.experimental import pallas as pl
from jax.experimental.pallas import tpu as pltpu
from jax.experimental.pallas import tpu_sc as plsc

N = 10000
E = 320000
IN_CH = 128
HID = 64
R = 3

f32 = jnp.float32
i32 = jnp.int32

NC = 2
NS = 16
NW = NC * NS
EPT = E // NW
CH = 80
NCHUNK = EPT // CH
EPS = E // NS
NCHUNK_C = EPS // CH
ROWS_T = (R + 1) * N
ACC_PT = 600
ACC_XT = N - NS * ACC_PT
ZR = 200
CNT_PT = 1920
CNT_PAD = NS * CNT_PT

_mesh = plsc.VectorSubcoreMesh(core_axis_name="c", subcore_axis_name="s")
_sc_params = pltpu.CompilerParams(use_tc_tiling_on_sc=False)


def _prep_body(et_hbm, dst_hbm, w_hbm,
               eb, db, sidxb, onesb, cbuf, wbuf, cnt_sh, lsem, ssem, osem):
    s = lax.axis_index("s")
    c = lax.axis_index("c")
    wid = s * NC + c

    z16 = jnp.zeros((16,), f32)
    one16 = jnp.ones((16,), f32)
    for k in range(CH // 16):
        onesb[pl.ds(k * 16, 16)] = one16

    def zcb(i, _):
        cbuf[pl.ds(i * 16, 16)] = z16
        return 0

    lax.fori_loop(0, CNT_PT // 16, zcb, 0)
    pltpu.sync_copy(cbuf, cnt_sh.at[pl.ds(s * CNT_PT, CNT_PT)])
    plsc.subcore_barrier()

    base1 = s * EPS

    def issue_loads1(ci, b):
        off = base1 + ci * CH
        pltpu.async_copy(et_hbm.at[pl.ds(off, CH)], eb.at[b], lsem)
        pltpu.async_copy(dst_hbm.at[pl.ds(off, CH)], db.at[b], lsem)

    def wait_loads():
        pltpu.make_async_copy(et_hbm.at[pl.ds(0, CH)], eb.at[0], lsem).wait()
        pltpu.make_async_copy(dst_hbm.at[pl.ds(0, CH)], db.at[0], lsem).wait()

    def wait_scat():
        pltpu.make_async_copy(
            onesb, cnt_sh.at[pl.ds(0, CH)], ssem).wait()

    issue_loads1(0, 0)

    def body1(i, _):
        bi = i % 2
        bn = 1 - bi
        wait_loads()

        @pl.when(i >= 2)
        def _():
            wait_scat()

        for k in range(CH // 16):
            sl = pl.ds(k * 16, 16)
            sidxb[bi, sl] = db[bi, sl] * R + eb[bi, sl]
        pltpu.async_copy(onesb, cnt_sh.at[sidxb.at[bi]], ssem, add=True)
        nx = jnp.minimum(i + 1, NCHUNK_C - 1)
        issue_loads1(nx, bn)
        return 0

    lax.fori_loop(0, NCHUNK_C, body1, 0)
    wait_scat()
    wait_scat()
    wait_loads()
    plsc.subcore_barrier()

    pltpu.sync_copy(cnt_sh.at[pl.ds(s * CNT_PT, CNT_PT)], cbuf)

    def invb(i, _):
        sl = pl.ds(i * 16, 16)
        cbuf[sl] = 1.0 / jnp.maximum(cbuf[sl], 1.0)
        return 0

    lax.fori_loop(0, CNT_PT // 16, invb, 0)
    pltpu.sync_copy(cbuf, cnt_sh.at[pl.ds(s * CNT_PT, CNT_PT)])
    plsc.subcore_barrier()

    base3 = wid * EPT

    def issue_loads3(ci, b):
        off = base3 + ci * CH
        pltpu.async_copy(et_hbm.at[pl.ds(off, CH)], eb.at[b], lsem)
        pltpu.async_copy(dst_hbm.at[pl.ds(off, CH)], db.at[b], lsem)

    issue_loads3(0, 0)

    def body3(i, _):
        bi = i % 2
        bn = 1 - bi
        wait_loads()
        for k in range(CH // 16):
            sl = pl.ds(k * 16, 16)
            sidxb[bi, sl] = db[bi, sl] * R + eb[bi, sl]

        @pl.when(i >= 2)
        def _():
            pltpu.make_async_copy(
                wbuf.at[0], w_hbm.at[pl.ds(0, CH)], osem).wait()

        pltpu.sync_copy(cnt_sh.at[sidxb.at[bi]], wbuf.at[bi])
        off = base3 + i * CH
        pltpu.async_copy(wbuf.at[bi], w_hbm.at[pl.ds(off, CH)], osem)
        nx = jnp.minimum(i + 1, NCHUNK - 1)
        issue_loads3(nx, bn)
        return 0

    lax.fori_loop(0, NCHUNK, body3, 0)
    for _ in range(2):
        pltpu.make_async_copy(
            wbuf.at[0], w_hbm.at[pl.ds(0, CH)], osem).wait()
    wait_loads()


_prep_call = pl.kernel(
    _prep_body,
    out_type=jax.ShapeDtypeStruct((E,), f32),
    mesh=_mesh,
    compiler_params=_sc_params,
    scratch_types=[
        pltpu.VMEM((2, CH), i32),
        pltpu.VMEM((2, CH), i32),
        pltpu.VMEM((2, CH), i32),
        pltpu.VMEM((CH,), f32),
        pltpu.VMEM((CNT_PT,), f32),
        pltpu.VMEM((2, CH), f32),
        pltpu.VMEM_SHARED((CNT_PAD,), f32),
        pltpu.SemaphoreType.DMA,
        pltpu.SemaphoreType.DMA,
        pltpu.SemaphoreType.DMA,
    ],
)


def _zero_acc(s, zb, acc_sh):
    def zacc(j, _):
        pltpu.sync_copy(zb, acc_sh.at[pl.ds(s * ACC_PT + j * ZR, ZR)])
        return 0

    lax.fori_loop(0, ACC_PT // ZR, zacc, 0)

    @pl.when(s == 0)
    def _():
        def zext(j, _):
            pltpu.sync_copy(zb, acc_sh.at[pl.ds(NS * ACC_PT + j * ZR, ZR)])
            return 0

        lax.fori_loop(0, ACC_XT // ZR, zext, 0)


def _wb_acc(c, s, acc_sh, accp):
    def wb(j, _):
        r0 = s * ACC_PT + j * ZR
        pltpu.sync_copy(acc_sh.at[pl.ds(r0, ZR)], accp.at[c, pl.ds(r0, ZR)])
        return 0

    lax.fori_loop(0, ACC_PT // ZR, wb, 0)

    @pl.when(s == 0)
    def _():
        def wbe(j, _):
            r0 = NS * ACC_PT + j * ZR
            pltpu.sync_copy(acc_sh.at[pl.ds(r0, ZR)],
                            accp.at[c, pl.ds(r0, ZR)])
            return 0

        lax.fori_loop(0, ACC_XT // ZR, wbe, 0)


def _scatter_body(t_hbm, et_hbm, src_hbm, dst_hbm, w_hbm, accp,
                  eb, sb, db, wbuf, gidxb, msgb, zb, acc_sh,
                  lsem, gsem, ssem):
    c = lax.axis_index("c")
    s = lax.axis_index("s")
    wid = s * NC + c
    ebase = wid * EPT

    z16 = jnp.zeros((16,), f32)

    def zrow(i, _):
        for j in range(HID // 16):
            zb[i, pl.ds(j * 16, 16)] = z16
        return 0

    lax.fori_loop(0, ZR, zrow, 0)
    _zero_acc(s, zb, acc_sh)
    plsc.subcore_barrier()

    def issue_loads(ci, b):
        off = ebase + ci * CH
        pltpu.async_copy(et_hbm.at[pl.ds(off, CH)], eb.at[b], lsem)
        pltpu.async_copy(src_hbm.at[pl.ds(off, CH)], sb.at[b], lsem)
        pltpu.async_copy(dst_hbm.at[pl.ds(off, CH)], db.at[b], lsem)
        pltpu.async_copy(w_hbm.at[pl.ds(off, CH)], wbuf.at[b], lsem)

    def wait_loads():
        pltpu.make_async_copy(et_hbm.at[pl.ds(0, CH)], eb.at[0], lsem).wait()
        pltpu.make_async_copy(src_hbm.at[pl.ds(0, CH)], sb.at[0], lsem).wait()
        pltpu.make_async_copy(dst_hbm.at[pl.ds(0, CH)], db.at[0], lsem).wait()
        pltpu.make_async_copy(
            w_hbm.at[pl.ds(0, CH)], wbuf.at[0], lsem).wait()

    def wait_scat():
        pltpu.make_async_copy(
            msgb.at[0], acc_sh.at[pl.ds(0, CH)], ssem).wait()

    def scale_and_scatter(bm, b3):
        for g in range(CH // 16):
            w16 = wbuf[b3, pl.ds(g * 16, 16)]
            for m in range(16):
                k = g * 16 + m
                wk = w16[m]
                for j in range(HID // 16):
                    sl = pl.ds(j * 16, 16)
                    msgb[bm, k, sl] = msgb[bm, k, sl] * wk
        pltpu.async_copy(msgb.at[bm], acc_sh.at[db.at[b3]], ssem, add=True)

    issue_loads(0, 0)

    def body(i, _):
        bi = i % 2
        bn = 1 - bi
        b3 = lax.rem(i, 3)
        p3 = lax.rem(i + 2, 3)
        wait_loads()
        for k in range(CH // 16):
            sl = pl.ds(k * 16, 16)
            gidxb[bi, sl] = sb[b3, sl] * (R + 1) + eb[b3, sl]

        @pl.when(i >= 2)
        def _():
            wait_scat()

        pltpu.async_copy(t_hbm.at[gidxb.at[bi]], msgb.at[bi], gsem)
        nx = jnp.minimum(i + 1, NCHUNK - 1)
        issue_loads(nx, lax.rem(i + 1, 3))

        @pl.when(i >= 1)
        def _():
            pltpu.make_async_copy(
                t_hbm.at[gidxb.at[bn]], msgb.at[bn], gsem).wait()
            scale_and_scatter(bn, p3)

        return 0

    lax.fori_loop(0, NCHUNK, body, 0)
    pltpu.make_async_copy(
        t_hbm.at[gidxb.at[(NCHUNK - 1) % 2]],
        msgb.at[(NCHUNK - 1) % 2], gsem).wait()
    scale_and_scatter((NCHUNK - 1) % 2, (NCHUNK - 1) % 3)
    wait_scat()
    wait_scat()
    wait_loads()
    plsc.subcore_barrier()
    _wb_acc(c, s, acc_sh, accp)


_scatter_call = pl.kernel(
    _scatter_body,
    out_type=jax.ShapeDtypeStruct((NC, N, HID), f32),
    mesh=_mesh,
    compiler_params=_sc_params,
    scratch_types=[
        pltpu.VMEM((3, CH), i32),
        pltpu.VMEM((3, CH), i32),
        pltpu.VMEM((3, CH), i32),
        pltpu.VMEM((3, CH), f32),
        pltpu.VMEM((2, CH), i32),
        pltpu.VMEM((2, CH, HID), f32),
        pltpu.VMEM((ZR, HID), f32),
        pltpu.VMEM_SHARED((N, HID), f32),
        pltpu.SemaphoreType.DMA,
        pltpu.SemaphoreType.DMA,
        pltpu.SemaphoreType.DMA,
    ],
)


def _edge_body(a_hbm, b_hbm, src_hbm, dst_hbm, h_hbm,
               sb, db, ga, gb, hbuf, lsem, gsem, osem):
    c = lax.axis_index("c")
    s = lax.axis_index("s")
    wid = s * NC + c
    ebase = wid * EPT

    def issue_loads(ci, b):
        off = ebase + ci * CH
        pltpu.async_copy(src_hbm.at[pl.ds(off, CH)], sb.at[b], lsem)
        pltpu.async_copy(dst_hbm.at[pl.ds(off, CH)], db.at[b], lsem)

    def wait_loads():
        pltpu.make_async_copy(src_hbm.at[pl.ds(0, CH)], sb.at[0], lsem).wait()
        pltpu.make_async_copy(dst_hbm.at[pl.ds(0, CH)], db.at[0], lsem).wait()

    def wait_gath():
        pltpu.make_async_copy(a_hbm.at[sb.at[0]], ga.at[0], gsem).wait()
        pltpu.make_async_copy(b_hbm.at[db.at[0]], gb.at[0], gsem).wait()

    def wait_out():
        pltpu.make_async_copy(
            hbuf.at[0], h_hbm.at[pl.ds(0, CH // 2)], osem).wait()

    def combine_and_write(i, b):
        for k in range(CH):
            for j in range(HID // 16):
                sl = pl.ds(j * 16, 16)
                dsl = pl.ds((k % 2) * HID + j * 16, 16)
                hbuf[b, k // 2, dsl] = jnp.maximum(
                    ga[b, k, sl] + gb[b, k, sl], 0.0)
        off2 = (ebase + i * CH) // 2
        pltpu.async_copy(hbuf.at[b], h_hbm.at[pl.ds(off2, CH // 2)], osem)

    issue_loads(0, 0)

    def body(i, _):
        bi = i % 2
        bn = 1 - bi
        wait_loads()

        @pl.when(i >= 2)
        def _():
            wait_out()

        pltpu.async_copy(a_hbm.at[sb.at[bi]], ga.at[bi], gsem)
        pltpu.async_copy(b_hbm.at[db.at[bi]], gb.at[bi], gsem)

        @pl.when(i >= 1)
        def _():
            wait_gath()

        nx = jnp.minimum(i + 1, NCHUNK - 1)
        issue_loads(nx, bn)

        @pl.when(i >= 1)
        def _():
            combine_and_write(i - 1, bn)

        return 0

    lax.fori_loop(0, NCHUNK, body, 0)
    blast = (NCHUNK - 1) % 2
    wait_gath()
    combine_and_write(NCHUNK - 1, blast)
    wait_out()
    wait_out()
    wait_loads()


_edge_call = pl.kernel(
    _edge_body,
    out_type=jax.ShapeDtypeStruct((E // 2, 2 * HID), f32),
    mesh=_mesh,
    compiler_params=_sc_params,
    scratch_types=[
        pltpu.VMEM((2, CH), i32),
        pltpu.VMEM((2, CH), i32),
        pltpu.VMEM((2, CH, HID), f32),
        pltpu.VMEM((2, CH, HID), f32),
        pltpu.VMEM((2, CH // 2, 2 * HID), f32),
        pltpu.SemaphoreType.DMA,
        pltpu.SemaphoreType.DMA,
        pltpu.SemaphoreType.DMA,
    ],
)


def _mm_body(x_ref, w_ref, b_ref, o_ref):
    o_ref[...] = (
        jnp.dot(x_ref[...], w_ref[...], preferred_element_type=f32)
        + b_ref[...]
    )


def _mm(xx, w, b, bn):
    m, k = xx.shape
    return pl.pallas_call(
        _mm_body,
        grid=(m // bn,),
        in_specs=[
            pl.BlockSpec((bn, k), lambda i: (i, 0)),
            pl.BlockSpec(w.shape, lambda i: (0, 0)),
            pl.BlockSpec(b.shape, lambda i: (0, 0)),
        ],
        out_specs=pl.BlockSpec((bn, w.shape[1]), lambda i: (i, 0)),
        out_shape=jax.ShapeDtypeStruct((m, w.shape[1]), f32),
    )(xx, w, b)


def _agg(t_ref, p0_ref, p1_ref):
    root = t_ref[...][:, R * HID:]
    return root + p0_ref[...] + p1_ref[...]


def _combine_mm_body(t_ref, p0_ref, p1_ref, w_ref, b_ref, xn_ref, tn_ref):
    x = jnp.maximum(_agg(t_ref, p0_ref, p1_ref), 0.0)
    xn_ref[...] = x
    tn_ref[...] = (
        jnp.dot(x, w_ref[...], preferred_element_type=f32) + b_ref[...]
    )


def _combine_mm(t, p0, p1, w, b, bn):
    return pl.pallas_call(
        _combine_mm_body,
        grid=(N // bn,),
        in_specs=[
            pl.BlockSpec((bn, (R + 1) * HID), lambda i: (i, 0)),
            pl.BlockSpec((bn, HID), lambda i: (i, 0)),
            pl.BlockSpec((bn, HID), lambda i: (i, 0)),
            pl.BlockSpec(w.shape, lambda i: (0, 0)),
            pl.BlockSpec(b.shape, lambda i: (0, 0)),
        ],
        out_specs=[
            pl.BlockSpec((bn, HID), lambda i: (i, 0)),
            pl.BlockSpec((bn, w.shape[1]), lambda i: (i, 0)),
        ],
        out_shape=[
            jax.ShapeDtypeStruct((N, HID), f32),
            jax.ShapeDtypeStruct((N, w.shape[1]), f32),
        ],
    )(t, p0, p1, w, b)


def _combine3_body(t_ref, p0_ref, p1_ref, x1_ref, w_ref, b_ref,
                   a_ref, bo_ref):
    x3 = _agg(t_ref, p0_ref, p1_ref) + x1_ref[...]
    ab = jnp.dot(x3, w_ref[...], preferred_element_type=f32) + b_ref[...]
    a_ref[...] = ab[:, :HID]
    bo_ref[...] = ab[:, HID:]


def _combine3(t, p0, p1, x1, w, b, bn):
    return pl.pallas_call(
        _combine3_body,
        grid=(N // bn,),
        in_specs=[
            pl.BlockSpec((bn, (R + 1) * HID), lambda i: (i, 0)),
            pl.BlockSpec((bn, HID), lambda i: (i, 0)),
            pl.BlockSpec((bn, HID), lambda i: (i, 0)),
            pl.BlockSpec((bn, HID), lambda i: (i, 0)),
            pl.BlockSpec(w.shape, lambda i: (0, 0)),
            pl.BlockSpec(b.shape, lambda i: (0, 0)),
        ],
        out_specs=[
            pl.BlockSpec((bn, HID), lambda i: (i, 0)),
            pl.BlockSpec((bn, HID), lambda i: (i, 0)),
        ],
        out_shape=[
            jax.ShapeDtypeStruct((N, HID), f32),
            jax.ShapeDtypeStruct((N, HID), f32),
        ],
    )(t, p0, p1, x1, w, b)


def _edge_mlp_body(h_ref, w2_ref, b2_ref, w3_ref, b3_ref, o_ref):
    h2 = jnp.maximum(
        jnp.dot(h_ref[...], w2_ref[...], preferred_element_type=f32)
        + b2_ref[...], 0.0
    )
    o_ref[...] = (
        jnp.dot(h2, w3_ref[...], preferred_element_type=f32) + b3_ref[...]
    )


def _edge_mlp(h, w2, b2, w3, b3, be):
    m = E // 2
    return pl.pallas_call(
        _edge_mlp_body,
        grid=(m // be,),
        in_specs=[
            pl.BlockSpec((be, 2 * HID), lambda i: (i, 0)),
            pl.BlockSpec(w2.shape, lambda i: (0, 0)),
            pl.BlockSpec(b2.shape, lambda i: (0, 0)),
            pl.BlockSpec(w3.shape, lambda i: (0, 0)),
            pl.BlockSpec(b3.shape, lambda i: (0, 0)),
        ],
        out_specs=pl.BlockSpec((be, 6), lambda i: (i, 0)),
        out_shape=jax.ShapeDtypeStruct((m, 6), f32),
    )(h, w2, b2, w3, b3)


def kernel(x, edge_index, edge_type, W1_rel, W1_root, b1, W2_rel, W2_root,
           b2, W3_rel, W3_root, b3, Wc1, bc1, Wc2, bc2, Wc3, bc3):
    src = edge_index[0]
    dst = edge_index[1]

    def wcat(w_rel, w_root, b):
        w = jnp.concatenate([w_rel[0], w_rel[1], w_rel[2], w_root], axis=1)
        bias = jnp.concatenate([jnp.zeros((R * HID,), f32), b])[None]
        return w, bias

    w1, bias1 = wcat(W1_rel, W1_root, b1)
    w2, bias2 = wcat(W2_rel, W2_root, b2)
    w3, bias3 = wcat(W3_rel, W3_root, b3)
    wab = jnp.concatenate([Wc1[:HID], Wc1[HID:]], axis=1)
    bab = jnp.concatenate([bc1, jnp.zeros((HID,), f32)])[None]

    bn = 1000
    wedge = _prep_call(edge_type, dst)

    t1 = _mm(x, w1, bias1, bn)
    p1 = _scatter_call(t1.reshape(ROWS_T, HID), edge_type, src, dst, wedge)
    x1, t2 = _combine_mm(t1, p1[0], p1[1], w2, bias2, bn)
    p2 = _scatter_call(t2.reshape(ROWS_T, HID), edge_type, src, dst, wedge)
    _, t3 = _combine_mm(t2, p2[0], p2[1], w3, bias3, bn)
    p3 = _scatter_call(t3.reshape(ROWS_T, HID), edge_type, src, dst, wedge)
    a, bnode = _combine3(t3, p3[0], p3[1], x1, wab, bab, bn)

    w2d = jnp.zeros((2 * HID, 2 * (HID // 2)), f32)
    w2d = w2d.at[:HID, :HID // 2].set(Wc2).at[HID:, HID // 2:].set(Wc2)
    b2d = jnp.concatenate([bc2, bc2])[None]
    w3d = jnp.zeros((2 * (HID // 2), 6), f32)
    w3d = w3d.at[:HID // 2, :3].set(Wc3).at[HID // 2:, 3:].set(Wc3)
    b3d = jnp.concatenate([bc3, bc3])[None]

    h = _edge_call(a, bnode, src, dst)
    out2 = _edge_mlp(h, w2d, b2d, w3d, b3d, 8000)
    return out2.reshape(E, 3)


# speedup vs baseline: 10.6973x; 1.0566x over previous
import jax
import jax.numpy as jnp
from jax import lax
from jax.experimental import pallas as pl
from jax.experimental.pallas import tpu as pltpu
from jax.experimental.pallas import tpu_sc as plsc

N = 10000
E = 320000
IN_CH = 128
HID = 64
R = 3

f32 = jnp.float32
i32 = jnp.int32

NC = 2
NS = 16
NW = NC * NS
EPT = E // NW
CH = 80
NCHUNK = EPT // CH
EPS = E // NS
CHP = 128
NCH1 = (EPS - 32) // CHP
TL1 = EPS - NCH1 * CHP
NCH3 = (EPT - 16) // CHP
TL3 = EPT - NCH3 * CHP
ROWS_T = (R + 1) * N
ACC_PT = 600
ACC_XT = N - NS * ACC_PT
ZR = 200
CNT_PT = 1920
CNT_PAD = NS * CNT_PT

_mesh = plsc.VectorSubcoreMesh(core_axis_name="c", subcore_axis_name="s")
_sc_params = pltpu.CompilerParams(use_tc_tiling_on_sc=False)


def _prep_body(et_hbm, dst_hbm, w_hbm,
               eb, db, sidxb, sidxt, onesb, cbuf, wbuf, cnt_sh,
               lsem, ssem, osem):
    s = lax.axis_index("s")
    c = lax.axis_index("c")
    wid = s * NC + c

    z16 = jnp.zeros((16,), f32)
    one16 = jnp.ones((16,), f32)
    for k in range(CHP // 16):
        onesb[pl.ds(k * 16, 16)] = one16

    def zcb(i, _):
        cbuf[pl.ds(i * 16, 16)] = z16
        return 0

    lax.fori_loop(0, CNT_PT // 16, zcb, 0)
    pltpu.sync_copy(cbuf, cnt_sh.at[pl.ds(s * CNT_PT, CNT_PT)])
    plsc.subcore_barrier()

    base1 = s * EPS

    def issue_loads1(ci, b):
        off = base1 + ci * CHP
        pltpu.async_copy(et_hbm.at[pl.ds(off, CHP)], eb.at[b], lsem)
        pltpu.async_copy(dst_hbm.at[pl.ds(off, CHP)], db.at[b], lsem)

    def wait_loads():
        pltpu.make_async_copy(et_hbm.at[pl.ds(0, CHP)], eb.at[0], lsem).wait()
        pltpu.make_async_copy(dst_hbm.at[pl.ds(0, CHP)], db.at[0], lsem).wait()

    def wait_scat():
        pltpu.make_async_copy(
            onesb, cnt_sh.at[pl.ds(0, CHP)], ssem).wait()

    issue_loads1(0, 0)

    def body1(i, _):
        bi = i % 2
        bn = 1 - bi
        wait_loads()

        @pl.when(i >= 2)
        def _():
            wait_scat()

        for k in range(CHP // 16):
            sl = pl.ds(k * 16, 16)
            sidxb[bi, sl] = db[bi, sl] * R + eb[bi, sl]
        pltpu.async_copy(onesb, cnt_sh.at[sidxb.at[bi]], ssem, add=True)
        nx = jnp.minimum(i + 1, NCH1 - 1)
        issue_loads1(nx, bn)
        return 0

    lax.fori_loop(0, NCH1, body1, 0)
    wait_scat()
    wait_scat()
    wait_loads()
    toff = base1 + NCH1 * CHP
    pltpu.sync_copy(et_hbm.at[pl.ds(toff, TL1)], eb.at[0, pl.ds(0, TL1)])
    pltpu.sync_copy(dst_hbm.at[pl.ds(toff, TL1)], db.at[0, pl.ds(0, TL1)])
    for k in range(TL1 // 16):
        sl = pl.ds(k * 16, 16)
        sidxt[0, sl] = db[0, sl] * R + eb[0, sl]
    pltpu.sync_copy(onesb.at[pl.ds(0, TL1)], cnt_sh.at[sidxt.at[0]],
                    add=True)
    plsc.subcore_barrier()

    pltpu.sync_copy(cnt_sh.at[pl.ds(s * CNT_PT, CNT_PT)], cbuf)

    def invb(i, _):
        sl = pl.ds(i * 16, 16)
        cbuf[sl] = 1.0 / jnp.maximum(cbuf[sl], 1.0)
        return 0

    lax.fori_loop(0, CNT_PT // 16, invb, 0)
    pltpu.sync_copy(cbuf, cnt_sh.at[pl.ds(s * CNT_PT, CNT_PT)])
    plsc.subcore_barrier()

    base3 = wid * EPT

    def issue_loads3(ci, b):
        off = base3 + ci * CHP
        pltpu.async_copy(et_hbm.at[pl.ds(off, CHP)], eb.at[b], lsem)
        pltpu.async_copy(dst_hbm.at[pl.ds(off, CHP)], db.at[b], lsem)

    issue_loads3(0, 0)

    def body3(i, _):
        bi = i % 2
        bn = 1 - bi
        wait_loads()
        for k in range(CHP // 16):
            sl = pl.ds(k * 16, 16)
            sidxb[bi, sl] = db[bi, sl] * R + eb[bi, sl]

        @pl.when(i >= 2)
        def _():
            pltpu.make_async_copy(
                wbuf.at[0], w_hbm.at[pl.ds(0, CHP)], osem).wait()

        pltpu.sync_copy(cnt_sh.at[sidxb.at[bi]], wbuf.at[bi])
        off = base3 + i * CHP
        pltpu.async_copy(wbuf.at[bi], w_hbm.at[pl.ds(off, CHP)], osem)
        nx = jnp.minimum(i + 1, NCH3 - 1)
        issue_loads3(nx, bn)
        return 0

    lax.fori_loop(0, NCH3, body3, 0)
    for _ in range(2):
        pltpu.make_async_copy(
            wbuf.at[0], w_hbm.at[pl.ds(0, CHP)], osem).wait()
    wait_loads()
    toff3 = base3 + NCH3 * CHP
    pltpu.sync_copy(et_hbm.at[pl.ds(toff3, TL3)], eb.at[0, pl.ds(0, TL3)])
    pltpu.sync_copy(dst_hbm.at[pl.ds(toff3, TL3)], db.at[0, pl.ds(0, TL3)])
    sl = pl.ds(0, 16)
    sidxt[0, sl] = db[0, sl] * R + eb[0, sl]
    pltpu.sync_copy(cnt_sh.at[sidxt.at[0, pl.ds(0, TL3)]],
                    wbuf.at[0, pl.ds(0, TL3)])
    pltpu.sync_copy(wbuf.at[0, pl.ds(0, TL3)], w_hbm.at[pl.ds(toff3, TL3)])


_prep_call = pl.kernel(
    _prep_body,
    out_type=jax.ShapeDtypeStruct((E,), f32),
    mesh=_mesh,
    compiler_params=_sc_params,
    scratch_types=[
        pltpu.VMEM((2, CHP), i32),
        pltpu.VMEM((2, CHP), i32),
        pltpu.VMEM((2, CHP), i32),
        pltpu.VMEM((1, 32), i32),
        pltpu.VMEM((CHP,), f32),
        pltpu.VMEM((CNT_PT,), f32),
        pltpu.VMEM((2, CHP), f32),
        pltpu.VMEM_SHARED((CNT_PAD,), f32),
        pltpu.SemaphoreType.DMA,
        pltpu.SemaphoreType.DMA,
        pltpu.SemaphoreType.DMA,
    ],
)


def _zero_acc(s, zb, acc_sh):
    def zacc(j, _):
        pltpu.sync_copy(zb, acc_sh.at[pl.ds(s * ACC_PT + j * ZR, ZR)])
        return 0

    lax.fori_loop(0, ACC_PT // ZR, zacc, 0)

    @pl.when(s == 0)
    def _():
        def zext(j, _):
            pltpu.sync_copy(zb, acc_sh.at[pl.ds(NS * ACC_PT + j * ZR, ZR)])
            return 0

        lax.fori_loop(0, ACC_XT // ZR, zext, 0)


def _wb_acc(c, s, acc_sh, accp):
    def wb(j, _):
        r0 = s * ACC_PT + j * ZR
        pltpu.sync_copy(acc_sh.at[pl.ds(r0, ZR)], accp.at[c, pl.ds(r0, ZR)])
        return 0

    lax.fori_loop(0, ACC_PT // ZR, wb, 0)

    @pl.when(s == 0)
    def _():
        def wbe(j, _):
            r0 = NS * ACC_PT + j * ZR
            pltpu.sync_copy(acc_sh.at[pl.ds(r0, ZR)],
                            accp.at[c, pl.ds(r0, ZR)])
            return 0

        lax.fori_loop(0, ACC_XT // ZR, wbe, 0)


def _scatter_body(t_hbm, et_hbm, src_hbm, dst_hbm, w_hbm, accp,
                  eb, sb, db, wbuf, gidxb, msgb, zb, acc_sh,
                  lsem, gsem, ssem):
    c = lax.axis_index("c")
    s = lax.axis_index("s")
    wid = s * NC + c
    ebase = wid * EPT

    z16 = jnp.zeros((16,), f32)

    def zrow(i, _):
        for j in range(HID // 16):
            zb[i, pl.ds(j * 16, 16)] = z16
        return 0

    lax.fori_loop(0, ZR, zrow, 0)
    _zero_acc(s, zb, acc_sh)
    plsc.subcore_barrier()

    def issue_loads(ci, b):
        off = ebase + ci * CH
        pltpu.async_copy(et_hbm.at[pl.ds(off, CH)], eb.at[b], lsem)
        pltpu.async_copy(src_hbm.at[pl.ds(off, CH)], sb.at[b], lsem)
        pltpu.async_copy(dst_hbm.at[pl.ds(off, CH)], db.at[b], lsem)
        pltpu.async_copy(w_hbm.at[pl.ds(off, CH)], wbuf.at[b], lsem)

    def wait_loads():
        pltpu.make_async_copy(et_hbm.at[pl.ds(0, CH)], eb.at[0], lsem).wait()
        pltpu.make_async_copy(src_hbm.at[pl.ds(0, CH)], sb.at[0], lsem).wait()
        pltpu.make_async_copy(dst_hbm.at[pl.ds(0, CH)], db.at[0], lsem).wait()
        pltpu.make_async_copy(
            w_hbm.at[pl.ds(0, CH)], wbuf.at[0], lsem).wait()

    def wait_scat():
        pltpu.make_async_copy(
            msgb.at[0], acc_sh.at[pl.ds(0, CH)], ssem).wait()

    def scale_and_scatter(bm, b3):
        for g in range(CH // 16):
            w16 = wbuf[b3, pl.ds(g * 16, 16)]
            for m in range(16):
                k = g * 16 + m
                wk = w16[m]
                for j in range(HID // 16):
                    sl = pl.ds(j * 16, 16)
                    msgb[bm, k, sl] = msgb[bm, k, sl] * wk
        pltpu.async_copy(msgb.at[bm], acc_sh.at[db.at[b3]], ssem, add=True)

    issue_loads(0, 0)

    def body(i, _):
        bi = i % 2
        bn = 1 - bi
        b3 = lax.rem(i, 3)
        p3 = lax.rem(i + 2, 3)
        wait_loads()
        for k in range(CH // 16):
            sl = pl.ds(k * 16, 16)
            gidxb[bi, sl] = sb[b3, sl] * (R + 1) + eb[b3, sl]

        @pl.when(i >= 2)
        def _():
            wait_scat()

        pltpu.async_copy(t_hbm.at[gidxb.at[bi]], msgb.at[bi], gsem)
        nx = jnp.minimum(i + 1, NCHUNK - 1)
        issue_loads(nx, lax.rem(i + 1, 3))

        @pl.when(i >= 1)
        def _():
            pltpu.make_async_copy(
                t_hbm.at[gidxb.at[bn]], msgb.at[bn], gsem).wait()
            scale_and_scatter(bn, p3)

        return 0

    lax.fori_loop(0, NCHUNK, body, 0)
    pltpu.make_async_copy(
        t_hbm.at[gidxb.at[(NCHUNK - 1) % 2]],
        msgb.at[(NCHUNK - 1) % 2], gsem).wait()
    scale_and_scatter((NCHUNK - 1) % 2, (NCHUNK - 1) % 3)
    wait_scat()
    wait_scat()
    wait_loads()
    plsc.subcore_barrier()
    _wb_acc(c, s, acc_sh, accp)


_scatter_call = pl.kernel(
    _scatter_body,
    out_type=jax.ShapeDtypeStruct((NC, N, HID), f32),
    mesh=_mesh,
    compiler_params=_sc_params,
    scratch_types=[
        pltpu.VMEM((3, CH), i32),
        pltpu.VMEM((3, CH), i32),
        pltpu.VMEM((3, CH), i32),
        pltpu.VMEM((3, CH), f32),
        pltpu.VMEM((2, CH), i32),
        pltpu.VMEM((2, CH, HID), f32),
        pltpu.VMEM((ZR, HID), f32),
        pltpu.VMEM_SHARED((N, HID), f32),
        pltpu.SemaphoreType.DMA,
        pltpu.SemaphoreType.DMA,
        pltpu.SemaphoreType.DMA,
    ],
)


def _edge_body(a_hbm, b_hbm, src_hbm, dst_hbm, h_hbm,
               sb, db, ga, gb, hbuf, lsem, gsem, osem):
    c = lax.axis_index("c")
    s = lax.axis_index("s")
    wid = s * NC + c
    ebase = wid * EPT

    def issue_loads(ci, b):
        off = ebase + ci * CH
        pltpu.async_copy(src_hbm.at[pl.ds(off, CH)], sb.at[b], lsem)
        pltpu.async_copy(dst_hbm.at[pl.ds(off, CH)], db.at[b], lsem)

    def wait_loads():
        pltpu.make_async_copy(src_hbm.at[pl.ds(0, CH)], sb.at[0], lsem).wait()
        pltpu.make_async_copy(dst_hbm.at[pl.ds(0, CH)], db.at[0], lsem).wait()

    def wait_gath():
        pltpu.make_async_copy(a_hbm.at[sb.at[0]], ga.at[0], gsem).wait()
        pltpu.make_async_copy(b_hbm.at[db.at[0]], gb.at[0], gsem).wait()

    def wait_out():
        pltpu.make_async_copy(
            hbuf.at[0], h_hbm.at[pl.ds(0, CH // 2)], osem).wait()

    def combine_and_write(i, b):
        for k in range(CH):
            for j in range(HID // 16):
                sl = pl.ds(j * 16, 16)
                dsl = pl.ds((k % 2) * HID + j * 16, 16)
                hbuf[b, k // 2, dsl] = jnp.maximum(
                    ga[b, k, sl] + gb[b, k, sl], 0.0)
        off2 = (ebase + i * CH) // 2
        pltpu.async_copy(hbuf.at[b], h_hbm.at[pl.ds(off2, CH // 2)], osem)

    issue_loads(0, 0)

    def body(i, _):
        bi = i % 2
        bn = 1 - bi
        wait_loads()

        @pl.when(i >= 2)
        def _():
            wait_out()

        pltpu.async_copy(a_hbm.at[sb.at[bi]], ga.at[bi], gsem)
        pltpu.async_copy(b_hbm.at[db.at[bi]], gb.at[bi], gsem)

        @pl.when(i >= 1)
        def _():
            wait_gath()

        nx = jnp.minimum(i + 1, NCHUNK - 1)
        issue_loads(nx, bn)

        @pl.when(i >= 1)
        def _():
            combine_and_write(i - 1, bn)

        return 0

    lax.fori_loop(0, NCHUNK, body, 0)
    blast = (NCHUNK - 1) % 2
    wait_gath()
    combine_and_write(NCHUNK - 1, blast)
    wait_out()
    wait_out()
    wait_loads()


_edge_call = pl.kernel(
    _edge_body,
    out_type=jax.ShapeDtypeStruct((E // 2, 2 * HID), f32),
    mesh=_mesh,
    compiler_params=_sc_params,
    scratch_types=[
        pltpu.VMEM((2, CH), i32),
        pltpu.VMEM((2, CH), i32),
        pltpu.VMEM((2, CH, HID), f32),
        pltpu.VMEM((2, CH, HID), f32),
        pltpu.VMEM((2, CH // 2, 2 * HID), f32),
        pltpu.SemaphoreType.DMA,
        pltpu.SemaphoreType.DMA,
        pltpu.SemaphoreType.DMA,
    ],
)


def _mm_body(x_ref, w_ref, b_ref, o_ref):
    o_ref[...] = (
        jnp.dot(x_ref[...], w_ref[...], preferred_element_type=f32)
        + b_ref[...]
    )


def _mm(xx, w, b, bn):
    m, k = xx.shape
    return pl.pallas_call(
        _mm_body,
        grid=(m // bn,),
        in_specs=[
            pl.BlockSpec((bn, k), lambda i: (i, 0)),
            pl.BlockSpec(w.shape, lambda i: (0, 0)),
            pl.BlockSpec(b.shape, lambda i: (0, 0)),
        ],
        out_specs=pl.BlockSpec((bn, w.shape[1]), lambda i: (i, 0)),
        out_shape=jax.ShapeDtypeStruct((m, w.shape[1]), f32),
    )(xx, w, b)


def _agg(t_ref, p0_ref, p1_ref):
    root = t_ref[...][:, R * HID:]
    return root + p0_ref[...] + p1_ref[...]


def _combine_mm_body(t_ref, p0_ref, p1_ref, w_ref, b_ref, xn_ref, tn_ref):
    x = jnp.maximum(_agg(t_ref, p0_ref, p1_ref), 0.0)
    xn_ref[...] = x
    tn_ref[...] = (
        jnp.dot(x, w_ref[...], preferred_element_type=f32) + b_ref[...]
    )


def _combine_mm(t, p0, p1, w, b, bn):
    return pl.pallas_call(
        _combine_mm_body,
        grid=(N // bn,),
        in_specs=[
            pl.BlockSpec((bn, (R + 1) * HID), lambda i: (i, 0)),
            pl.BlockSpec((bn, HID), lambda i: (i, 0)),
            pl.BlockSpec((bn, HID), lambda i: (i, 0)),
            pl.BlockSpec(w.shape, lambda i: (0, 0)),
            pl.BlockSpec(b.shape, lambda i: (0, 0)),
        ],
        out_specs=[
            pl.BlockSpec((bn, HID), lambda i: (i, 0)),
            pl.BlockSpec((bn, w.shape[1]), lambda i: (i, 0)),
        ],
        out_shape=[
            jax.ShapeDtypeStruct((N, HID), f32),
            jax.ShapeDtypeStruct((N, w.shape[1]), f32),
        ],
    )(t, p0, p1, w, b)


def _combine3_body(t_ref, p0_ref, p1_ref, x1_ref, w_ref, b_ref,
                   a_ref, bo_ref):
    x3 = _agg(t_ref, p0_ref, p1_ref) + x1_ref[...]
    ab = jnp.dot(x3, w_ref[...], preferred_element_type=f32) + b_ref[...]
    a_ref[...] = ab[:, :HID]
    bo_ref[...] = ab[:, HID:]


def _combine3(t, p0, p1, x1, w, b, bn):
    return pl.pallas_call(
        _combine3_body,
        grid=(N // bn,),
        in_specs=[
            pl.BlockSpec((bn, (R + 1) * HID), lambda i: (i, 0)),
            pl.BlockSpec((bn, HID), lambda i: (i, 0)),
            pl.BlockSpec((bn, HID), lambda i: (i, 0)),
            pl.BlockSpec((bn, HID), lambda i: (i, 0)),
            pl.BlockSpec(w.shape, lambda i: (0, 0)),
            pl.BlockSpec(b.shape, lambda i: (0, 0)),
        ],
        out_specs=[
            pl.BlockSpec((bn, HID), lambda i: (i, 0)),
            pl.BlockSpec((bn, HID), lambda i: (i, 0)),
        ],
        out_shape=[
            jax.ShapeDtypeStruct((N, HID), f32),
            jax.ShapeDtypeStruct((N, HID), f32),
        ],
    )(t, p0, p1, x1, w, b)


def _edge_mlp_body(h_ref, w2_ref, b2_ref, w3_ref, b3_ref, o_ref):
    h2 = jnp.maximum(
        jnp.dot(h_ref[...], w2_ref[...], preferred_element_type=f32)
        + b2_ref[...], 0.0
    )
    o_ref[...] = (
        jnp.dot(h2, w3_ref[...], preferred_element_type=f32) + b3_ref[...]
    )


def _edge_mlp(h, w2, b2, w3, b3, be):
    m = E // 2
    return pl.pallas_call(
        _edge_mlp_body,
        grid=(m // be,),
        in_specs=[
            pl.BlockSpec((be, 2 * HID), lambda i: (i, 0)),
            pl.BlockSpec(w2.shape, lambda i: (0, 0)),
            pl.BlockSpec(b2.shape, lambda i: (0, 0)),
            pl.BlockSpec(w3.shape, lambda i: (0, 0)),
            pl.BlockSpec(b3.shape, lambda i: (0, 0)),
        ],
        out_specs=pl.BlockSpec((be, 6), lambda i: (i, 0)),
        out_shape=jax.ShapeDtypeStruct((m, 6), f32),
    )(h, w2, b2, w3, b3)


def kernel(x, edge_index, edge_type, W1_rel, W1_root, b1, W2_rel, W2_root,
           b2, W3_rel, W3_root, b3, Wc1, bc1, Wc2, bc2, Wc3, bc3):
    src = edge_index[0]
    dst = edge_index[1]

    def wcat(w_rel, w_root, b):
        w = jnp.concatenate([w_rel[0], w_rel[1], w_rel[2], w_root], axis=1)
        bias = jnp.concatenate([jnp.zeros((R * HID,), f32), b])[None]
        return w, bias

    w1, bias1 = wcat(W1_rel, W1_root, b1)
    w2, bias2 = wcat(W2_rel, W2_root, b2)
    w3, bias3 = wcat(W3_rel, W3_root, b3)
    wab = jnp.concatenate([Wc1[:HID], Wc1[HID:]], axis=1)
    bab = jnp.concatenate([bc1, jnp.zeros((HID,), f32)])[None]

    bn = 1000
    wedge = _prep_call(edge_type, dst)

    t1 = _mm(x, w1, bias1, bn)
    p1 = _scatter_call(t1.reshape(ROWS_T, HID), edge_type, src, dst, wedge)
    x1, t2 = _combine_mm(t1, p1[0], p1[1], w2, bias2, bn)
    p2 = _scatter_call(t2.reshape(ROWS_T, HID), edge_type, src, dst, wedge)
    _, t3 = _combine_mm(t2, p2[0], p2[1], w3, bias3, bn)
    p3 = _scatter_call(t3.reshape(ROWS_T, HID), edge_type, src, dst, wedge)
    a, bnode = _combine3(t3, p3[0], p3[1], x1, wab, bab, bn)

    w2d = jnp.zeros((2 * HID, 2 * (HID // 2)), f32)
    w2d = w2d.at[:HID, :HID // 2].set(Wc2).at[HID:, HID // 2:].set(Wc2)
    b2d = jnp.concatenate([bc2, bc2])[None]
    w3d = jnp.zeros((2 * (HID // 2), 6), f32)
    w3d = w3d.at[:HID // 2, :3].set(Wc3).at[HID // 2:, 3:].set(Wc3)
    b3d = jnp.concatenate([bc3, bc3])[None]

    h = _edge_call(a, bnode, src, dst)
    out2 = _edge_mlp(h, w2d, b2d, w3d, b3d, 8000)
    return out2.reshape(E, 3)


# speedup vs baseline: 11.3369x; 1.0598x over previous
import jax
import jax.numpy as jnp
from jax import lax
from jax.experimental import pallas as pl
from jax.experimental.pallas import tpu as pltpu
from jax.experimental.pallas import tpu_sc as plsc

N = 10000
E = 320000
IN_CH = 128
HID = 64
R = 3

f32 = jnp.float32
i32 = jnp.int32

NC = 2
NS = 16
NW = NC * NS
EPT = E // NW
CH = 80
NCHUNK = EPT // CH
EPS = E // NS
CHP = 128
NCH1 = (EPS - 32) // CHP
TL1 = EPS - NCH1 * CHP
NCH3 = (EPT - 16) // CHP
TL3 = EPT - NCH3 * CHP
ROWS_T = (R + 1) * N
ACC_PT = 600
ACC_XT = N - NS * ACC_PT
ZR = 200
CNT_PT = 1920
CNT_PAD = NS * CNT_PT

_mesh = plsc.VectorSubcoreMesh(core_axis_name="c", subcore_axis_name="s")
_sc_params = pltpu.CompilerParams(use_tc_tiling_on_sc=False)


def _prep_body(et_hbm, dst_hbm, w_hbm,
               eb, db, sidxb, sidxt, onesb, cbuf, wbuf, cnt_sh,
               lsem, ssem, osem):
    s = lax.axis_index("s")
    c = lax.axis_index("c")
    wid = s * NC + c

    z16 = jnp.zeros((16,), f32)
    one16 = jnp.ones((16,), f32)
    for k in range(CHP // 16):
        onesb[pl.ds(k * 16, 16)] = one16

    def zcb(i, _):
        cbuf[pl.ds(i * 16, 16)] = z16
        return 0

    lax.fori_loop(0, CNT_PT // 16, zcb, 0)
    pltpu.sync_copy(cbuf, cnt_sh.at[pl.ds(s * CNT_PT, CNT_PT)])
    plsc.subcore_barrier()

    base1 = s * EPS

    def issue_loads1(ci, b):
        off = base1 + ci * CHP
        pltpu.async_copy(et_hbm.at[pl.ds(off, CHP)], eb.at[b], lsem)
        pltpu.async_copy(dst_hbm.at[pl.ds(off, CHP)], db.at[b], lsem)

    def wait_loads():
        pltpu.make_async_copy(et_hbm.at[pl.ds(0, CHP)], eb.at[0], lsem).wait()
        pltpu.make_async_copy(dst_hbm.at[pl.ds(0, CHP)], db.at[0], lsem).wait()

    def wait_scat():
        pltpu.make_async_copy(
            onesb, cnt_sh.at[pl.ds(0, CHP)], ssem).wait()

    issue_loads1(0, 0)

    def body1(i, _):
        bi = i % 2
        bn = 1 - bi
        wait_loads()

        @pl.when(i >= 2)
        def _():
            wait_scat()

        for k in range(CHP // 16):
            sl = pl.ds(k * 16, 16)
            sidxb[bi, sl] = db[bi, sl] * R + eb[bi, sl]
        pltpu.async_copy(onesb, cnt_sh.at[sidxb.at[bi]], ssem, add=True)
        nx = jnp.minimum(i + 1, NCH1 - 1)
        issue_loads1(nx, bn)
        return 0

    lax.fori_loop(0, NCH1, body1, 0)
    wait_scat()
    wait_scat()
    wait_loads()
    toff = base1 + NCH1 * CHP
    pltpu.sync_copy(et_hbm.at[pl.ds(toff, TL1)], eb.at[0, pl.ds(0, TL1)])
    pltpu.sync_copy(dst_hbm.at[pl.ds(toff, TL1)], db.at[0, pl.ds(0, TL1)])
    for k in range(TL1 // 16):
        sl = pl.ds(k * 16, 16)
        sidxt[0, sl] = db[0, sl] * R + eb[0, sl]
    pltpu.sync_copy(onesb.at[pl.ds(0, TL1)], cnt_sh.at[sidxt.at[0]],
                    add=True)
    plsc.subcore_barrier()

    pltpu.sync_copy(cnt_sh.at[pl.ds(s * CNT_PT, CNT_PT)], cbuf)

    def invb(i, _):
        sl = pl.ds(i * 16, 16)
        cbuf[sl] = 1.0 / jnp.maximum(cbuf[sl], 1.0)
        return 0

    lax.fori_loop(0, CNT_PT // 16, invb, 0)
    pltpu.sync_copy(cbuf, cnt_sh.at[pl.ds(s * CNT_PT, CNT_PT)])
    plsc.subcore_barrier()

    base3 = wid * EPT

    def issue_loads3(ci, b):
        off = base3 + ci * CHP
        pltpu.async_copy(et_hbm.at[pl.ds(off, CHP)], eb.at[b], lsem)
        pltpu.async_copy(dst_hbm.at[pl.ds(off, CHP)], db.at[b], lsem)

    issue_loads3(0, 0)

    def body3(i, _):
        bi = i % 2
        bn = 1 - bi
        wait_loads()
        for k in range(CHP // 16):
            sl = pl.ds(k * 16, 16)
            sidxb[bi, sl] = db[bi, sl] * R + eb[bi, sl]

        @pl.when(i >= 2)
        def _():
            pltpu.make_async_copy(
                wbuf.at[0], w_hbm.at[pl.ds(0, CHP)], osem).wait()

        pltpu.sync_copy(cnt_sh.at[sidxb.at[bi]], wbuf.at[bi])
        off = base3 + i * CHP
        pltpu.async_copy(wbuf.at[bi], w_hbm.at[pl.ds(off, CHP)], osem)
        nx = jnp.minimum(i + 1, NCH3 - 1)
        issue_loads3(nx, bn)
        return 0

    lax.fori_loop(0, NCH3, body3, 0)
    for _ in range(2):
        pltpu.make_async_copy(
            wbuf.at[0], w_hbm.at[pl.ds(0, CHP)], osem).wait()
    wait_loads()
    toff3 = base3 + NCH3 * CHP
    pltpu.sync_copy(et_hbm.at[pl.ds(toff3, TL3)], eb.at[0, pl.ds(0, TL3)])
    pltpu.sync_copy(dst_hbm.at[pl.ds(toff3, TL3)], db.at[0, pl.ds(0, TL3)])
    sl = pl.ds(0, 16)
    sidxt[0, sl] = db[0, sl] * R + eb[0, sl]
    pltpu.sync_copy(cnt_sh.at[sidxt.at[0, pl.ds(0, TL3)]],
                    wbuf.at[0, pl.ds(0, TL3)])
    pltpu.sync_copy(wbuf.at[0, pl.ds(0, TL3)], w_hbm.at[pl.ds(toff3, TL3)])


_prep_call = pl.kernel(
    _prep_body,
    out_type=jax.ShapeDtypeStruct((E,), f32),
    mesh=_mesh,
    compiler_params=_sc_params,
    scratch_types=[
        pltpu.VMEM((2, CHP), i32),
        pltpu.VMEM((2, CHP), i32),
        pltpu.VMEM((2, CHP), i32),
        pltpu.VMEM((1, 32), i32),
        pltpu.VMEM((CHP,), f32),
        pltpu.VMEM((CNT_PT,), f32),
        pltpu.VMEM((2, CHP), f32),
        pltpu.VMEM_SHARED((CNT_PAD,), f32),
        pltpu.SemaphoreType.DMA,
        pltpu.SemaphoreType.DMA,
        pltpu.SemaphoreType.DMA,
    ],
)


def _zero_acc(s, zb, acc_sh):
    def zacc(j, _):
        pltpu.sync_copy(zb, acc_sh.at[pl.ds(s * ACC_PT + j * ZR, ZR)])
        return 0

    lax.fori_loop(0, ACC_PT // ZR, zacc, 0)

    @pl.when(s == 0)
    def _():
        def zext(j, _):
            pltpu.sync_copy(zb, acc_sh.at[pl.ds(NS * ACC_PT + j * ZR, ZR)])
            return 0

        lax.fori_loop(0, ACC_XT // ZR, zext, 0)


def _wb_acc(c, s, acc_sh, accp):
    def wb(j, _):
        r0 = s * ACC_PT + j * ZR
        pltpu.sync_copy(acc_sh.at[pl.ds(r0, ZR)], accp.at[c, pl.ds(r0, ZR)])
        return 0

    lax.fori_loop(0, ACC_PT // ZR, wb, 0)

    @pl.when(s == 0)
    def _():
        def wbe(j, _):
            r0 = NS * ACC_PT + j * ZR
            pltpu.sync_copy(acc_sh.at[pl.ds(r0, ZR)],
                            accp.at[c, pl.ds(r0, ZR)])
            return 0

        lax.fori_loop(0, ACC_XT // ZR, wbe, 0)


def _scatter_body(t_hbm, et_hbm, src_hbm, dst_hbm, w_hbm, accp,
                  eb, sb, db, wbuf, gidxb, msgb, zb, acc_sh,
                  lsem, gsem, ssem):
    c = lax.axis_index("c")
    s = lax.axis_index("s")
    wid = s * NC + c
    ebase = wid * EPT

    z16 = jnp.zeros((16,), f32)

    def zrow(i, _):
        for j in range(HID // 16):
            zb[i, pl.ds(j * 16, 16)] = z16
        return 0

    lax.fori_loop(0, ZR, zrow, 0)
    _zero_acc(s, zb, acc_sh)
    plsc.subcore_barrier()

    def issue_loads(ci, b):
        off = ebase + ci * CH
        pltpu.async_copy(et_hbm.at[pl.ds(off, CH)], eb.at[b], lsem)
        pltpu.async_copy(src_hbm.at[pl.ds(off, CH)], sb.at[b], lsem)
        pltpu.async_copy(dst_hbm.at[pl.ds(off, CH)], db.at[b], lsem)
        pltpu.async_copy(w_hbm.at[pl.ds(off, CH)], wbuf.at[b], lsem)

    def wait_loads():
        pltpu.make_async_copy(et_hbm.at[pl.ds(0, CH)], eb.at[0], lsem).wait()
        pltpu.make_async_copy(src_hbm.at[pl.ds(0, CH)], sb.at[0], lsem).wait()
        pltpu.make_async_copy(dst_hbm.at[pl.ds(0, CH)], db.at[0], lsem).wait()
        pltpu.make_async_copy(
            w_hbm.at[pl.ds(0, CH)], wbuf.at[0], lsem).wait()

    def wait_scat():
        pltpu.make_async_copy(
            msgb.at[0], acc_sh.at[pl.ds(0, CH)], ssem).wait()

    def scale_and_scatter(bm, b3):
        for g in range(CH // 16):
            w16 = wbuf[b3, pl.ds(g * 16, 16)]
            for m in range(16):
                k = g * 16 + m
                wk = w16[m]
                for j in range(HID // 16):
                    sl = pl.ds(j * 16, 16)
                    msgb[bm, k, sl] = msgb[bm, k, sl] * wk
        pltpu.async_copy(msgb.at[bm], acc_sh.at[db.at[b3]], ssem, add=True)

    issue_loads(0, 0)

    def body(i, _):
        bi = i % 2
        bn = 1 - bi
        b3 = lax.rem(i, 3)
        p3 = lax.rem(i + 2, 3)
        wait_loads()
        for k in range(CH // 16):
            sl = pl.ds(k * 16, 16)
            gidxb[bi, sl] = sb[b3, sl] * (R + 1) + eb[b3, sl]

        @pl.when(i >= 2)
        def _():
            wait_scat()

        pltpu.async_copy(t_hbm.at[gidxb.at[bi]], msgb.at[bi], gsem)
        nx = jnp.minimum(i + 1, NCHUNK - 1)
        issue_loads(nx, lax.rem(i + 1, 3))

        @pl.when(i >= 1)
        def _():
            pltpu.make_async_copy(
                t_hbm.at[gidxb.at[bn]], msgb.at[bn], gsem).wait()
            scale_and_scatter(bn, p3)

        return 0

    lax.fori_loop(0, NCHUNK, body, 0)
    pltpu.make_async_copy(
        t_hbm.at[gidxb.at[(NCHUNK - 1) % 2]],
        msgb.at[(NCHUNK - 1) % 2], gsem).wait()
    scale_and_scatter((NCHUNK - 1) % 2, (NCHUNK - 1) % 3)
    wait_scat()
    wait_scat()
    wait_loads()
    plsc.subcore_barrier()
    _wb_acc(c, s, acc_sh, accp)


_scatter_call = pl.kernel(
    _scatter_body,
    out_type=jax.ShapeDtypeStruct((NC, N, HID), f32),
    mesh=_mesh,
    compiler_params=_sc_params,
    scratch_types=[
        pltpu.VMEM((3, CH), i32),
        pltpu.VMEM((3, CH), i32),
        pltpu.VMEM((3, CH), i32),
        pltpu.VMEM((3, CH), f32),
        pltpu.VMEM((2, CH), i32),
        pltpu.VMEM((2, CH, HID), f32),
        pltpu.VMEM((ZR, HID), f32),
        pltpu.VMEM_SHARED((N, HID), f32),
        pltpu.SemaphoreType.DMA,
        pltpu.SemaphoreType.DMA,
        pltpu.SemaphoreType.DMA,
    ],
)


def _edge_body(a_hbm, b_hbm, src_hbm, dst_hbm, h_hbm,
               sb, db, ga, gb, hbuf, lsem, gsem, osem):
    c = lax.axis_index("c")
    s = lax.axis_index("s")
    wid = s * NC + c
    ebase = wid * EPT

    def issue_loads(ci, b):
        off = ebase + ci * CH
        pltpu.async_copy(src_hbm.at[pl.ds(off, CH)], sb.at[b], lsem)
        pltpu.async_copy(dst_hbm.at[pl.ds(off, CH)], db.at[b], lsem)

    def wait_loads():
        pltpu.make_async_copy(src_hbm.at[pl.ds(0, CH)], sb.at[0], lsem).wait()
        pltpu.make_async_copy(dst_hbm.at[pl.ds(0, CH)], db.at[0], lsem).wait()

    def wait_gath():
        pltpu.make_async_copy(a_hbm.at[sb.at[0]], ga.at[0], gsem).wait()
        pltpu.make_async_copy(b_hbm.at[db.at[0]], gb.at[0], gsem).wait()

    half = wid // NS
    cq = half * HID
    rbase = ebase - half * (E // 2)

    def wait_out():
        pltpu.make_async_copy(
            hbuf.at[0],
            h_hbm.at[pl.ds(0, CH), pl.ds(0, HID)], osem).wait()

    def combine_and_write(i, b):
        for k in range(CH):
            for j in range(HID // 16):
                sl = pl.ds(j * 16, 16)
                hbuf[b, k, sl] = jnp.maximum(
                    ga[b, k, sl] + gb[b, k, sl], 0.0)
        r0 = rbase + i * CH
        pltpu.async_copy(
            hbuf.at[b], h_hbm.at[pl.ds(r0, CH), pl.ds(cq, HID)], osem)

    issue_loads(0, 0)

    def body(i, _):
        bi = i % 2
        bn = 1 - bi
        wait_loads()

        @pl.when(i >= 2)
        def _():
            wait_out()

        pltpu.async_copy(a_hbm.at[sb.at[bi]], ga.at[bi], gsem)
        pltpu.async_copy(b_hbm.at[db.at[bi]], gb.at[bi], gsem)

        @pl.when(i >= 1)
        def _():
            wait_gath()

        nx = jnp.minimum(i + 1, NCHUNK - 1)
        issue_loads(nx, bn)

        @pl.when(i >= 1)
        def _():
            combine_and_write(i - 1, bn)

        return 0

    lax.fori_loop(0, NCHUNK, body, 0)
    blast = (NCHUNK - 1) % 2
    wait_gath()
    combine_and_write(NCHUNK - 1, blast)
    wait_out()
    wait_out()
    wait_loads()


_edge_call = pl.kernel(
    _edge_body,
    out_type=jax.ShapeDtypeStruct((E // 2, 2 * HID), f32),
    mesh=_mesh,
    compiler_params=_sc_params,
    scratch_types=[
        pltpu.VMEM((2, CH), i32),
        pltpu.VMEM((2, CH), i32),
        pltpu.VMEM((2, CH, HID), f32),
        pltpu.VMEM((2, CH, HID), f32),
        pltpu.VMEM((2, CH, HID), f32),
        pltpu.SemaphoreType.DMA,
        pltpu.SemaphoreType.DMA,
        pltpu.SemaphoreType.DMA,
    ],
)


def _mm_body(x_ref, w_ref, b_ref, o_ref):
    o_ref[...] = (
        jnp.dot(x_ref[...], w_ref[...], preferred_element_type=f32)
        + b_ref[...]
    )


def _mm(xx, w, b, bn):
    m, k = xx.shape
    return pl.pallas_call(
        _mm_body,
        grid=(m // bn,),
        in_specs=[
            pl.BlockSpec((bn, k), lambda i: (i, 0)),
            pl.BlockSpec(w.shape, lambda i: (0, 0)),
            pl.BlockSpec(b.shape, lambda i: (0, 0)),
        ],
        out_specs=pl.BlockSpec((bn, w.shape[1]), lambda i: (i, 0)),
        out_shape=jax.ShapeDtypeStruct((m, w.shape[1]), f32),
    )(xx, w, b)


def _agg(t_ref, p0_ref, p1_ref):
    root = t_ref[...][:, R * HID:]
    return root + p0_ref[...] + p1_ref[...]


def _combine_mm_body(t_ref, p0_ref, p1_ref, w_ref, b_ref, xn_ref, tn_ref):
    x = jnp.maximum(_agg(t_ref, p0_ref, p1_ref), 0.0)
    xn_ref[...] = x
    tn_ref[...] = (
        jnp.dot(x, w_ref[...], preferred_element_type=f32) + b_ref[...]
    )


def _combine_mm(t, p0, p1, w, b, bn):
    return pl.pallas_call(
        _combine_mm_body,
        grid=(N // bn,),
        in_specs=[
            pl.BlockSpec((bn, (R + 1) * HID), lambda i: (i, 0)),
            pl.BlockSpec((bn, HID), lambda i: (i, 0)),
            pl.BlockSpec((bn, HID), lambda i: (i, 0)),
            pl.BlockSpec(w.shape, lambda i: (0, 0)),
            pl.BlockSpec(b.shape, lambda i: (0, 0)),
        ],
        out_specs=[
            pl.BlockSpec((bn, HID), lambda i: (i, 0)),
            pl.BlockSpec((bn, w.shape[1]), lambda i: (i, 0)),
        ],
        out_shape=[
            jax.ShapeDtypeStruct((N, HID), f32),
            jax.ShapeDtypeStruct((N, w.shape[1]), f32),
        ],
    )(t, p0, p1, w, b)


def _combine3_body(t_ref, p0_ref, p1_ref, x1_ref, w_ref, b_ref,
                   a_ref, bo_ref):
    x3 = _agg(t_ref, p0_ref, p1_ref) + x1_ref[...]
    ab = jnp.dot(x3, w_ref[...], preferred_element_type=f32) + b_ref[...]
    a_ref[...] = ab[:, :HID]
    bo_ref[...] = ab[:, HID:]


def _combine3(t, p0, p1, x1, w, b, bn):
    return pl.pallas_call(
        _combine3_body,
        grid=(N // bn,),
        in_specs=[
            pl.BlockSpec((bn, (R + 1) * HID), lambda i: (i, 0)),
            pl.BlockSpec((bn, HID), lambda i: (i, 0)),
            pl.BlockSpec((bn, HID), lambda i: (i, 0)),
            pl.BlockSpec((bn, HID), lambda i: (i, 0)),
            pl.BlockSpec(w.shape, lambda i: (0, 0)),
            pl.BlockSpec(b.shape, lambda i: (0, 0)),
        ],
        out_specs=[
            pl.BlockSpec((bn, HID), lambda i: (i, 0)),
            pl.BlockSpec((bn, HID), lambda i: (i, 0)),
        ],
        out_shape=[
            jax.ShapeDtypeStruct((N, HID), f32),
            jax.ShapeDtypeStruct((N, HID), f32),
        ],
    )(t, p0, p1, x1, w, b)


def _edge_mlp_body(h_ref, w2_ref, b2_ref, w3_ref, b3_ref, o_ref):
    q = pl.program_id(1)
    hh = h_ref[...]
    hq = jnp.where(q == 0, hh[:, :HID], hh[:, HID:])
    h2 = jnp.maximum(
        jnp.dot(hq, w2_ref[...], preferred_element_type=f32)
        + b2_ref[...], 0.0
    )
    o_ref[...] = (
        jnp.dot(h2, w3_ref[...], preferred_element_type=f32) + b3_ref[...]
    )


def _edge_mlp(h, w2, b2, w3, b3, be):
    m = E // 2
    nb = m // be
    return pl.pallas_call(
        _edge_mlp_body,
        grid=(nb, 2),
        in_specs=[
            pl.BlockSpec((be, 2 * HID), lambda i, q: (i, 0)),
            pl.BlockSpec(w2.shape, lambda i, q: (0, 0)),
            pl.BlockSpec(b2.shape, lambda i, q: (0, 0)),
            pl.BlockSpec(w3.shape, lambda i, q: (0, 0)),
            pl.BlockSpec(b3.shape, lambda i, q: (0, 0)),
        ],
        out_specs=pl.BlockSpec((be, 3), lambda i, q: (q * nb + i, 0)),
        out_shape=jax.ShapeDtypeStruct((E, 3), f32),
    )(h, w2, b2, w3, b3)


def kernel(x, edge_index, edge_type, W1_rel, W1_root, b1, W2_rel, W2_root,
           b2, W3_rel, W3_root, b3, Wc1, bc1, Wc2, bc2, Wc3, bc3):
    src = edge_index[0]
    dst = edge_index[1]

    def wcat(w_rel, w_root, b):
        w = jnp.concatenate([w_rel[0], w_rel[1], w_rel[2], w_root], axis=1)
        bias = jnp.concatenate([jnp.zeros((R * HID,), f32), b])[None]
        return w, bias

    w1, bias1 = wcat(W1_rel, W1_root, b1)
    w2, bias2 = wcat(W2_rel, W2_root, b2)
    w3, bias3 = wcat(W3_rel, W3_root, b3)
    wab = jnp.concatenate([Wc1[:HID], Wc1[HID:]], axis=1)
    bab = jnp.concatenate([bc1, jnp.zeros((HID,), f32)])[None]

    bn = 1000
    wedge = _prep_call(edge_type, dst)

    t1 = _mm(x, w1, bias1, bn)
    p1 = _scatter_call(t1.reshape(ROWS_T, HID), edge_type, src, dst, wedge)
    x1, t2 = _combine_mm(t1, p1[0], p1[1], w2, bias2, bn)
    p2 = _scatter_call(t2.reshape(ROWS_T, HID), edge_type, src, dst, wedge)
    _, t3 = _combine_mm(t2, p2[0], p2[1], w3, bias3, bn)
    p3 = _scatter_call(t3.reshape(ROWS_T, HID), edge_type, src, dst, wedge)
    a, bnode = _combine3(t3, p3[0], p3[1], x1, wab, bab, bn)

    h = _edge_call(a, bnode, src, dst)
    return _edge_mlp(h, Wc2, bc2[None], Wc3, bc3[None], 8000)
